# Initial kernel scaffold; baseline (speedup 1.0000x reference)
import jax
import jax.numpy as jnp
from jax.experimental import pallas as pl


def kernel(seq_samples, structure, s_emb, W_e, b_e, enc_W1, enc_b1, enc_W2, enc_b2, dec_W1, dec_b1, dec_W2, dec_b2, W_out, b_out):
    raise NotImplementedError("write your pallas kernel here")


# speedup vs baseline: 10.1467x; 10.1467x over previous
import functools

import jax
import jax.numpy as jnp
from jax import lax
from jax.experimental import pallas as pl
from jax.experimental.pallas import tpu as pltpu
from jax.experimental.pallas import tpu_sc as plsc

B, L, K, H, A, NRBF, NL = 4, 512, 48, 128, 21, 16, 3
N = B * L
E_TOT = N * K
NW = 32
ROWS_W = E_TOT // NW
CHUNK = 128
NCH = ROWS_W // CHUNK
NT = 128

_SIGMA = (22.0 - 2.0) / NRBF


def _ln(x):
    mu = x.mean(-1, keepdims=True)
    var = ((x - mu) ** 2).mean(-1, keepdims=True)
    return (x - mu) / jnp.sqrt(var + 1e-5)


RT = 128


def _centers_row():
    return 2.0 + lax.broadcasted_iota(jnp.int32, (1, NRBF), 1).astype(
        jnp.float32) * (20.0 / (NRBF - 1))


def _stage_a_body(ca_s_ref, ca_t_ref, gq_ref, vqc_ref, gidx_ref, dnb_ref,
                  isig_ref):
    b = pl.program_id(0)
    cs = ca_s_ref[0]
    ct = ca_t_ref[0]
    dx = cs[:, 0:1] - ct[0:1, :]
    dy = cs[:, 1:2] - ct[1:2, :]
    dz = cs[:, 2:3] - ct[2:3, :]
    d = jnp.sqrt(dx * dx + dy * dy + dz * dz + 1e-6)
    iot = lax.broadcasted_iota(jnp.int32, (RT, L), 1)
    centers = _centers_row()
    vq_col = vqc_ref[0:1, :].reshape(NRBF, 1)
    cq = vqc_ref[1, 0]
    dcols, icols, scols = [], [], []
    for _ in range(K):
        m = jnp.min(d, axis=1, keepdims=True)
        im = jnp.min(jnp.where(d == m, iot, L), axis=1, keepdims=True)
        d = jnp.where(iot == im, jnp.float32(jnp.inf), d)
        dcols.append(m)
        icols.append(im)
        r = jnp.exp(-(((m - centers) / _SIGMA) ** 2))
        t = jnp.dot(r, gq_ref[...], preferred_element_type=jnp.float32)
        var = jnp.sum(t * r, axis=1, keepdims=True)
        var = var + 2.0 * jnp.dot(r, vq_col,
                                  preferred_element_type=jnp.float32) + cq
        scols.append(1.0 / jnp.sqrt(var + 1e-5))
    gidx_ref[0] = jnp.concatenate(icols, axis=1) + b * L
    dnb_ref[0] = jnp.concatenate(dcols, axis=1)
    isig_ref[0] = jnp.concatenate(scols, axis=1)


def _stage_a(ca_s, ca_t, gq, vqc):
    return pl.pallas_call(
        _stage_a_body,
        grid=(B, L // RT),
        in_specs=[
            pl.BlockSpec((1, RT, 8), lambda b, t: (b, t, 0)),
            pl.BlockSpec((1, 8, L), lambda b, t: (b, 0, 0)),
            pl.BlockSpec((NRBF, NRBF), lambda b, t: (0, 0)),
            pl.BlockSpec((8, NRBF), lambda b, t: (0, 0)),
        ],
        out_specs=[
            pl.BlockSpec((1, RT, K), lambda b, t: (b, t, 0)),
            pl.BlockSpec((1, RT, K), lambda b, t: (b, t, 0)),
            pl.BlockSpec((1, RT, K), lambda b, t: (b, t, 0)),
        ],
        out_shape=[
            jax.ShapeDtypeStruct((B, L, K), jnp.int32),
            jax.ShapeDtypeStruct((B, L, K), jnp.float32),
            jax.ShapeDtypeStruct((B, L, K), jnp.float32),
        ],
    )(ca_s, ca_t, gq, vqc)


def _prep_body(seq_ref, t_ref, out_ref):
    sc = seq_ref[0]
    iot = lax.broadcasted_iota(jnp.int32, (L, 32), 1)
    oh = (sc == iot).astype(jnp.float32)
    out_ref[0, 0] = jnp.dot(oh, t_ref[0],
                            preferred_element_type=jnp.float32)


def _prep(seq3, tpad):
    return pl.pallas_call(
        _prep_body,
        grid=(NL, B),
        in_specs=[
            pl.BlockSpec((1, L, 1), lambda l, b: (b, 0, 0)),
            pl.BlockSpec((1, 32, H), lambda l, b: (l, 0, 0)),
        ],
        out_specs=pl.BlockSpec((1, 1, L, H), lambda l, b: (l, b, 0, 0)),
        out_shape=jax.ShapeDtypeStruct((NL, B, L, H), jnp.float32),
    )(seq3, tpad)


def _layer_body(has_g, has_hs, has_p, *refs):
    i = 0
    if has_g:
        h_ref, g_ref = refs[0], refs[1]
        i = 2
    dnb_ref, isig_ref, m_ref, cb_ref, w1a_ref, w2_ref = refs[i:i + 6]
    i += 6
    if has_p:
        w1bn_ref = refs[i]
        i += 1
    if has_hs:
        hsn_ref = refs[i]
        i += 1
    hn_ref = refs[i]
    if has_p:
        p_ref = refs[i + 1]

    dnb = dnb_ref[...]
    isg = isig_ref[...]
    centers = _centers_row()
    rfeat = jnp.concatenate(
        [jnp.exp(-(((dnb[:, k:k + 1] - centers) / _SIGMA) ** 2))
         for k in range(K)], axis=0)
    stall = jnp.concatenate(
        [isg[:, k:k + 1] for k in range(K)], axis=0)
    rtall = jnp.dot(rfeat, m_ref[...],
                    preferred_element_type=jnp.float32)
    mfull = (rtall + cb_ref[0:1, :]) * stall
    if has_g:
        h = h_ref[...]
        hi = jnp.dot(h, w1a_ref[...],
                     preferred_element_type=jnp.float32) + cb_ref[1:2, :]
        g2 = g_ref[...].reshape(K * NT, H)
        mfull = mfull + g2 + jnp.concatenate([hi] * K, axis=0)
    else:
        mfull = mfull + cb_ref[1:2, :]
    gf = jax.nn.gelu(mfull)
    s = gf[0:NT]
    for k in range(1, K):
        s = s + gf[k * NT:(k + 1) * NT]
    s = s / jnp.float32(K)
    u = jnp.dot(s, w2_ref[...], preferred_element_type=jnp.float32)
    u = u + cb_ref[2:3, :]
    hn = _ln(h + u) if has_g else _ln(u)
    hn_ref[...] = hn
    if has_p:
        p = jnp.dot(hn, w1bn_ref[...], preferred_element_type=jnp.float32)
        if has_hs:
            p = p + hsn_ref[...]
        p_ref[...] = p


TPB = L // NT


def _layer(h, g, dnb, isig, m, cb, w1a, w2, w1bn, hsn):
    has_g = g is not None
    has_p = w1bn is not None
    has_hs = hsn is not None
    steps = N // NT
    node2 = pl.BlockSpec((NT, H), lambda i: (i, 0))
    const = lambda shape: pl.BlockSpec(shape, lambda i: (0, 0))
    in_specs, args = [], []
    if has_g:
        in_specs += [node2,
                     pl.BlockSpec((1, K, NT, H),
                                  lambda i: (i // TPB, 0, i % TPB, 0))]
        args += [h, g]
    in_specs += [pl.BlockSpec((NT, K), lambda i: (i, 0)),
                 pl.BlockSpec((NT, K), lambda i: (i, 0)),
                 const((NRBF, H)), const((8, H)), const((H, H)),
                 const((H, H))]
    args += [dnb, isig, m, cb, w1a, w2]
    if has_p:
        in_specs.append(const((H, H)))
        args.append(w1bn)
    if has_hs:
        in_specs.append(node2)
        args.append(hsn)
    out_specs = [node2]
    out_shape = [jax.ShapeDtypeStruct((N, H), jnp.float32)]
    if has_p:
        out_specs.append(node2)
        out_shape.append(jax.ShapeDtypeStruct((N, H), jnp.float32))
    outs = pl.pallas_call(
        functools.partial(_layer_body, has_g, has_hs, has_p),
        grid=(steps,),
        in_specs=in_specs,
        out_specs=out_specs,
        out_shape=out_shape,
    )(*args)
    return outs if has_p else (outs[0], None)


def _head_body(h_ref, seq_ref, wo_ref, bo_ref, out_ref):
    logits = jnp.dot(h_ref[...], wo_ref[...],
                     preferred_element_type=jnp.float32) + bo_ref[0:1, :]
    lane = lax.broadcasted_iota(jnp.int32, (L, H), 1)
    valid = lane < A
    neg = jnp.float32(-1e30)
    mx = jnp.max(jnp.where(valid, logits, neg), axis=1, keepdims=True)
    ex = jnp.where(valid, jnp.exp(logits - mx), 0.0)
    lse = jnp.log(jnp.sum(ex, axis=1, keepdims=True)) + mx
    sc = seq_ref[0]
    sel = jnp.sum(jnp.where(lane == sc, logits, 0.0), axis=1,
                  keepdims=True)
    val = jnp.sum(sel - lse) / jnp.float32(L)
    out_ref[...] = jnp.full((1, 1, H), val, jnp.float32)


def _head(h, seq3, wo, bo):
    return pl.pallas_call(
        _head_body,
        grid=(B,),
        in_specs=[
            pl.BlockSpec((L, H), lambda b: (b, 0)),
            pl.BlockSpec((1, L, 1), lambda b: (b, 0, 0)),
            pl.BlockSpec((H, H), lambda b: (0, 0)),
            pl.BlockSpec((8, H), lambda b: (0, 0)),
        ],
        out_specs=pl.BlockSpec((1, 1, H), lambda b: (b, 0, 0)),
        out_shape=jax.ShapeDtypeStruct((B, 1, H), jnp.float32),
    )(h, seq3, wo, bo)


def _sc_gather(table, gidx3):
    mesh = plsc.VectorSubcoreMesh(core_axis_name="c", subcore_axis_name="s")

    @functools.partial(
        pl.kernel, mesh=mesh,
        out_type=jax.ShapeDtypeStruct((NW, NCH, CHUNK, H), jnp.float32),
        scratch_types=[
            pltpu.VMEM((NCH, CHUNK), jnp.int32),
            pltpu.VMEM((CHUNK, H), jnp.float32),
            pltpu.SemaphoreType.DMA,
        ],
    )
    def k(table_hbm, idx_hbm, out_hbm, idx_v, rows_v, sem):
        w = lax.axis_index("s") * 2 + lax.axis_index("c")
        pltpu.sync_copy(idx_hbm.at[w], idx_v)

        def body(j, carry):
            pltpu.async_copy(table_hbm.at[idx_v.at[j]], rows_v, sem).wait()
            pltpu.sync_copy(rows_v, out_hbm.at[w, j])
            return carry

        lax.fori_loop(0, NCH, body, 0)

    return k(table, gidx3)


def _gather_edges(table, gidx3):
    out = _sc_gather(table, gidx3)
    return out.reshape(B, K, L, H)


def kernel(seq_samples, structure, s_emb, W_e, b_e, enc_W1, enc_b1, enc_W2,
           enc_b2, dec_W1, dec_b1, dec_W2, dec_b2, W_out, b_out):
    f32 = jnp.float32
    seq3 = seq_samples.astype(jnp.int32)[..., None]
    ca = structure[:, :, 1, :]
    ca_s = jnp.concatenate([ca, jnp.zeros((B, L, 5), f32)], axis=2)
    ca_t = jnp.swapaxes(ca_s, 1, 2)

    wbar = W_e.mean(axis=1, keepdims=True)
    wc = W_e - wbar
    bc = b_e - b_e.mean()
    gq = jnp.dot(wc, wc.T) / H
    vq = jnp.dot(wc, bc) / H
    vqc = jnp.zeros((8, NRBF), f32).at[0].set(vq).at[1, 0].set(
        jnp.dot(bc, bc) / H)

    gidx, dnb, isig = _stage_a(ca_s, ca_t, gq, vqc)
    gidx3 = gidx.transpose(0, 2, 1).reshape(NW, NCH, CHUNK)
    dnb = dnb.reshape(N, K)
    isig = isig.reshape(N, K)

    eW1a = [enc_W1[l][:H] for l in range(NL)]
    eW1b = [enc_W1[l][H:2 * H] for l in range(NL)]
    eM = [jnp.dot(wc, enc_W1[l][2 * H:]) for l in range(NL)]
    ec = [jnp.dot(bc, enc_W1[l][2 * H:]) for l in range(NL)]
    dW1a = [dec_W1[l][:H] for l in range(NL)]
    dW1b = [dec_W1[l][H:2 * H] for l in range(NL)]
    dT = jnp.stack([jnp.dot(s_emb, dec_W1[l][2 * H:3 * H])
                    for l in range(NL)])
    dM = [jnp.dot(wc, dec_W1[l][3 * H:]) for l in range(NL)]
    dc = [jnp.dot(bc, dec_W1[l][3 * H:]) for l in range(NL)]

    def cbpack(c, b1, b2):
        z = jnp.zeros((8, H), f32)
        return z.at[0].set(c).at[1].set(b1).at[2].set(b2)

    ecb = [cbpack(ec[l], enc_b1[l], enc_b2[l]) for l in range(NL)]
    dcb = [cbpack(dc[l], dec_b1[l], dec_b2[l]) for l in range(NL)]

    tpad = jnp.concatenate([dT, jnp.zeros((NL, 32 - A, H), f32)], axis=1)
    hs3 = _prep(seq3, tpad).reshape(NL, N, H)

    h, p = _layer(None, None, dnb, isig, eM[0], ecb[0], eW1a[0], enc_W2[0],
                  eW1b[1], None)
    g = _gather_edges(p, gidx3)
    h, p = _layer(h, g, dnb, isig, eM[1], ecb[1], eW1a[1], enc_W2[1],
                  eW1b[2], None)
    g = _gather_edges(p, gidx3)
    h, p = _layer(h, g, dnb, isig, eM[2], ecb[2], eW1a[2], enc_W2[2],
                  dW1b[0], hs3[0])
    g = _gather_edges(p, gidx3)
    h, p = _layer(h, g, dnb, isig, dM[0], dcb[0], dW1a[0], dec_W2[0],
                  dW1b[1], hs3[1])
    g = _gather_edges(p, gidx3)
    h, p = _layer(h, g, dnb, isig, dM[1], dcb[1], dW1a[1], dec_W2[1],
                  dW1b[2], hs3[2])
    g = _gather_edges(p, gidx3)
    h, _ = _layer(h, g, dnb, isig, dM[2], dcb[2], dW1a[2], dec_W2[2],
                  None, None)

    wo = jnp.concatenate([W_out, jnp.zeros((H, H - A), f32)], axis=1)
    bo = jnp.zeros((8, H), f32).at[0, :A].set(b_out)
    out = _head(h, seq3, wo, bo)
    return out[:, 0, 0]


# speedup vs baseline: 10.5304x; 1.0378x over previous
import functools

import jax
import jax.numpy as jnp
from jax import lax
from jax.experimental import pallas as pl
from jax.experimental.pallas import tpu as pltpu
from jax.experimental.pallas import tpu_sc as plsc

B, L, K, H, A, NRBF, NL = 4, 512, 48, 128, 21, 16, 3
N = B * L
E_TOT = N * K
NW = 32
ROWS_W = E_TOT // NW
CHUNK = 128
NCH = ROWS_W // CHUNK
NT = 128

_SIGMA = (22.0 - 2.0) / NRBF


def _ln(x):
    mu = x.mean(-1, keepdims=True)
    var = ((x - mu) ** 2).mean(-1, keepdims=True)
    return (x - mu) / jnp.sqrt(var + 1e-5)


RT = 128


def _centers_row():
    return 2.0 + lax.broadcasted_iota(jnp.int32, (1, NRBF), 1).astype(
        jnp.float32) * (20.0 / (NRBF - 1))


def _stage_a_body(ca_s_ref, ca_t_ref, gq_ref, vqc_ref, gidx_ref, dnb_ref,
                  isig_ref):
    b = pl.program_id(0)
    cs = ca_s_ref[0]
    ct = ca_t_ref[0]
    dx = cs[:, 0:1] - ct[0:1, :]
    dy = cs[:, 1:2] - ct[1:2, :]
    dz = cs[:, 2:3] - ct[2:3, :]
    d = jnp.sqrt(dx * dx + dy * dy + dz * dz + 1e-6)
    iot = lax.broadcasted_iota(jnp.int32, (RT, L), 1)
    centers = _centers_row()
    vq_col = vqc_ref[0:1, :].reshape(NRBF, 1)
    cq = vqc_ref[1, 0]
    dcols, icols = [], []
    for _ in range(K):
        m = jnp.min(d, axis=1, keepdims=True)
        im = jnp.min(jnp.where(d == m, iot, L), axis=1, keepdims=True)
        d = jnp.where(iot == im, jnp.float32(jnp.inf), d)
        dcols.append(m)
        icols.append(im)
    gidx_ref[0] = jnp.concatenate(icols, axis=1) + b * L
    dnb_ref[0] = jnp.concatenate(dcols, axis=1)
    dtall = jnp.concatenate(dcols, axis=0)
    r = jnp.exp(-(((dtall - centers) / _SIGMA) ** 2))
    t = jnp.dot(r, gq_ref[...], preferred_element_type=jnp.float32)
    var = jnp.sum(t * r, axis=1, keepdims=True)
    var = var + 2.0 * jnp.dot(r, vq_col,
                              preferred_element_type=jnp.float32) + cq
    istall = 1.0 / jnp.sqrt(var + 1e-5)
    isig_ref[0] = jnp.concatenate(
        [istall[k * RT:(k + 1) * RT] for k in range(K)], axis=1)


def _stage_a(ca_s, ca_t, gq, vqc):
    return pl.pallas_call(
        _stage_a_body,
        grid=(B, L // RT),
        in_specs=[
            pl.BlockSpec((1, RT, 8), lambda b, t: (b, t, 0)),
            pl.BlockSpec((1, 8, L), lambda b, t: (b, 0, 0)),
            pl.BlockSpec((NRBF, NRBF), lambda b, t: (0, 0)),
            pl.BlockSpec((8, NRBF), lambda b, t: (0, 0)),
        ],
        out_specs=[
            pl.BlockSpec((1, RT, K), lambda b, t: (b, t, 0)),
            pl.BlockSpec((1, RT, K), lambda b, t: (b, t, 0)),
            pl.BlockSpec((1, RT, K), lambda b, t: (b, t, 0)),
        ],
        out_shape=[
            jax.ShapeDtypeStruct((B, L, K), jnp.int32),
            jax.ShapeDtypeStruct((B, L, K), jnp.float32),
            jax.ShapeDtypeStruct((B, L, K), jnp.float32),
        ],
    )(ca_s, ca_t, gq, vqc)


def _prep_body(seq_ref, t_ref, out_ref):
    sc = seq_ref[0]
    iot = lax.broadcasted_iota(jnp.int32, (L, 32), 1)
    oh = (sc == iot).astype(jnp.float32)
    out_ref[0, 0] = jnp.dot(oh, t_ref[0],
                            preferred_element_type=jnp.float32)


def _prep(seq3, tpad):
    return pl.pallas_call(
        _prep_body,
        grid=(NL, B),
        in_specs=[
            pl.BlockSpec((1, L, 1), lambda l, b: (b, 0, 0)),
            pl.BlockSpec((1, 32, H), lambda l, b: (l, 0, 0)),
        ],
        out_specs=pl.BlockSpec((1, 1, L, H), lambda l, b: (l, b, 0, 0)),
        out_shape=jax.ShapeDtypeStruct((NL, B, L, H), jnp.float32),
    )(seq3, tpad)


def _layer_body(has_g, has_hs, has_p, *refs):
    i = 0
    if has_g:
        h_ref, g_ref = refs[0], refs[1]
        i = 2
    dnb_ref, isig_ref, m_ref, cb_ref, w1a_ref, w2_ref = refs[i:i + 6]
    i += 6
    if has_p:
        w1bn_ref = refs[i]
        i += 1
    if has_hs:
        hsn_ref = refs[i]
        i += 1
    hn_ref = refs[i]
    if has_p:
        p_ref = refs[i + 1]

    dnb = dnb_ref[...]
    isg = isig_ref[...]
    centers = _centers_row()
    rfeat = jnp.concatenate(
        [jnp.exp(-(((dnb[:, k:k + 1] - centers) / _SIGMA) ** 2))
         for k in range(K)], axis=0)
    stall = jnp.concatenate(
        [isg[:, k:k + 1] for k in range(K)], axis=0)
    rtall = jnp.dot(rfeat, m_ref[...],
                    preferred_element_type=jnp.float32)
    mfull = (rtall + cb_ref[0:1, :]) * stall
    if has_g:
        h = h_ref[...]
        hi = jnp.dot(h, w1a_ref[...],
                     preferred_element_type=jnp.float32) + cb_ref[1:2, :]
        g2 = g_ref[...].reshape(K * NT, H)
        mfull = mfull + g2 + jnp.concatenate([hi] * K, axis=0)
    else:
        mfull = mfull + cb_ref[1:2, :]
    gf = jax.nn.gelu(mfull)
    s = gf[0:NT]
    for k in range(1, K):
        s = s + gf[k * NT:(k + 1) * NT]
    s = s / jnp.float32(K)
    u = jnp.dot(s, w2_ref[...], preferred_element_type=jnp.float32)
    u = u + cb_ref[2:3, :]
    hn = _ln(h + u) if has_g else _ln(u)
    hn_ref[...] = hn
    if has_p:
        p = jnp.dot(hn, w1bn_ref[...], preferred_element_type=jnp.float32)
        if has_hs:
            p = p + hsn_ref[...]
        p_ref[...] = p


TPB = L // NT


def _layer(h, g, dnb, isig, m, cb, w1a, w2, w1bn, hsn):
    has_g = g is not None
    has_p = w1bn is not None
    has_hs = hsn is not None
    steps = N // NT
    node2 = pl.BlockSpec((NT, H), lambda i: (i, 0))
    const = lambda shape: pl.BlockSpec(shape, lambda i: (0, 0))
    in_specs, args = [], []
    if has_g:
        in_specs += [node2,
                     pl.BlockSpec((1, K, NT, H),
                                  lambda i: (i // TPB, 0, i % TPB, 0))]
        args += [h, g]
    in_specs += [pl.BlockSpec((NT, K), lambda i: (i, 0)),
                 pl.BlockSpec((NT, K), lambda i: (i, 0)),
                 const((NRBF, H)), const((8, H)), const((H, H)),
                 const((H, H))]
    args += [dnb, isig, m, cb, w1a, w2]
    if has_p:
        in_specs.append(const((H, H)))
        args.append(w1bn)
    if has_hs:
        in_specs.append(node2)
        args.append(hsn)
    out_specs = [node2]
    out_shape = [jax.ShapeDtypeStruct((N, H), jnp.float32)]
    if has_p:
        out_specs.append(node2)
        out_shape.append(jax.ShapeDtypeStruct((N, H), jnp.float32))
    outs = pl.pallas_call(
        functools.partial(_layer_body, has_g, has_hs, has_p),
        grid=(steps,),
        in_specs=in_specs,
        out_specs=out_specs,
        out_shape=out_shape,
    )(*args)
    return outs if has_p else (outs[0], None)


def _head_body(h_ref, seq_ref, wo_ref, bo_ref, out_ref):
    logits = jnp.dot(h_ref[...], wo_ref[...],
                     preferred_element_type=jnp.float32) + bo_ref[0:1, :]
    lane = lax.broadcasted_iota(jnp.int32, (L, H), 1)
    valid = lane < A
    neg = jnp.float32(-1e30)
    mx = jnp.max(jnp.where(valid, logits, neg), axis=1, keepdims=True)
    ex = jnp.where(valid, jnp.exp(logits - mx), 0.0)
    lse = jnp.log(jnp.sum(ex, axis=1, keepdims=True)) + mx
    sc = seq_ref[0]
    sel = jnp.sum(jnp.where(lane == sc, logits, 0.0), axis=1,
                  keepdims=True)
    val = jnp.sum(sel - lse) / jnp.float32(L)
    out_ref[...] = jnp.full((1, 1, H), val, jnp.float32)


def _head(h, seq3, wo, bo):
    return pl.pallas_call(
        _head_body,
        grid=(B,),
        in_specs=[
            pl.BlockSpec((L, H), lambda b: (b, 0)),
            pl.BlockSpec((1, L, 1), lambda b: (b, 0, 0)),
            pl.BlockSpec((H, H), lambda b: (0, 0)),
            pl.BlockSpec((8, H), lambda b: (0, 0)),
        ],
        out_specs=pl.BlockSpec((1, 1, H), lambda b: (b, 0, 0)),
        out_shape=jax.ShapeDtypeStruct((B, 1, H), jnp.float32),
    )(h, seq3, wo, bo)


def _sc_gather(table, gidx3):
    mesh = plsc.VectorSubcoreMesh(core_axis_name="c", subcore_axis_name="s")

    @functools.partial(
        pl.kernel, mesh=mesh,
        out_type=jax.ShapeDtypeStruct((NW, NCH, CHUNK, H), jnp.float32),
        scratch_types=[
            pltpu.VMEM((NCH, CHUNK), jnp.int32),
            pltpu.VMEM((CHUNK, H), jnp.float32),
            pltpu.VMEM((CHUNK, H), jnp.float32),
            pltpu.SemaphoreType.DMA,
            pltpu.SemaphoreType.DMA,
        ],
    )
    def k(table_hbm, idx_hbm, out_hbm, idx_v, rows0, rows1, sem0, sem1):
        w = lax.axis_index("s") * 2 + lax.axis_index("c")
        pltpu.sync_copy(idx_hbm.at[w], idx_v)
        bufs = (rows0, rows1)
        sems = (sem0, sem1)
        for t in range(2):
            pltpu.make_async_copy(table_hbm.at[idx_v.at[t]], bufs[t],
                                  sems[t]).start()

        def body(i, carry):
            j0 = 2 * i
            for t in range(2):
                j = j0 + t
                pltpu.make_async_copy(table_hbm.at[idx_v.at[j]], bufs[t],
                                      sems[t]).wait()
                pltpu.sync_copy(bufs[t], out_hbm.at[w, j])

                @pl.when(j + 2 < NCH)
                def _():
                    pltpu.make_async_copy(table_hbm.at[idx_v.at[j + 2]],
                                          bufs[t], sems[t]).start()
            return carry

        lax.fori_loop(0, NCH // 2, body, 0)

    return k(table, gidx3)


def _gather_edges(table, gidx3):
    out = _sc_gather(table, gidx3)
    return out.reshape(B, K, L, H)


def kernel(seq_samples, structure, s_emb, W_e, b_e, enc_W1, enc_b1, enc_W2,
           enc_b2, dec_W1, dec_b1, dec_W2, dec_b2, W_out, b_out):
    f32 = jnp.float32
    seq3 = seq_samples.astype(jnp.int32)[..., None]
    ca = structure[:, :, 1, :]
    ca_s = jnp.concatenate([ca, jnp.zeros((B, L, 5), f32)], axis=2)
    ca_t = jnp.swapaxes(ca_s, 1, 2)

    wbar = W_e.mean(axis=1, keepdims=True)
    wc = W_e - wbar
    bc = b_e - b_e.mean()
    gq = jnp.dot(wc, wc.T) / H
    vq = jnp.dot(wc, bc) / H
    vqc = jnp.zeros((8, NRBF), f32).at[0].set(vq).at[1, 0].set(
        jnp.dot(bc, bc) / H)

    gidx, dnb, isig = _stage_a(ca_s, ca_t, gq, vqc)
    gidx3 = gidx.transpose(0, 2, 1).reshape(NW, NCH, CHUNK)
    dnb = dnb.reshape(N, K)
    isig = isig.reshape(N, K)

    eW1a = [enc_W1[l][:H] for l in range(NL)]
    eW1b = [enc_W1[l][H:2 * H] for l in range(NL)]
    eM = [jnp.dot(wc, enc_W1[l][2 * H:]) for l in range(NL)]
    ec = [jnp.dot(bc, enc_W1[l][2 * H:]) for l in range(NL)]
    dW1a = [dec_W1[l][:H] for l in range(NL)]
    dW1b = [dec_W1[l][H:2 * H] for l in range(NL)]
    dT = jnp.stack([jnp.dot(s_emb, dec_W1[l][2 * H:3 * H])
                    for l in range(NL)])
    dM = [jnp.dot(wc, dec_W1[l][3 * H:]) for l in range(NL)]
    dc = [jnp.dot(bc, dec_W1[l][3 * H:]) for l in range(NL)]

    def cbpack(c, b1, b2):
        z = jnp.zeros((8, H), f32)
        return z.at[0].set(c).at[1].set(b1).at[2].set(b2)

    ecb = [cbpack(ec[l], enc_b1[l], enc_b2[l]) for l in range(NL)]
    dcb = [cbpack(dc[l], dec_b1[l], dec_b2[l]) for l in range(NL)]

    tpad = jnp.concatenate([dT, jnp.zeros((NL, 32 - A, H), f32)], axis=1)
    hs3 = _prep(seq3, tpad).reshape(NL, N, H)

    h, p = _layer(None, None, dnb, isig, eM[0], ecb[0], eW1a[0], enc_W2[0],
                  eW1b[1], None)
    g = _gather_edges(p, gidx3)
    h, p = _layer(h, g, dnb, isig, eM[1], ecb[1], eW1a[1], enc_W2[1],
                  eW1b[2], None)
    g = _gather_edges(p, gidx3)
    h, p = _layer(h, g, dnb, isig, eM[2], ecb[2], eW1a[2], enc_W2[2],
                  dW1b[0], hs3[0])
    g = _gather_edges(p, gidx3)
    h, p = _layer(h, g, dnb, isig, dM[0], dcb[0], dW1a[0], dec_W2[0],
                  dW1b[1], hs3[1])
    g = _gather_edges(p, gidx3)
    h, p = _layer(h, g, dnb, isig, dM[1], dcb[1], dW1a[1], dec_W2[1],
                  dW1b[2], hs3[2])
    g = _gather_edges(p, gidx3)
    h, _ = _layer(h, g, dnb, isig, dM[2], dcb[2], dW1a[2], dec_W2[2],
                  None, None)

    wo = jnp.concatenate([W_out, jnp.zeros((H, H - A), f32)], axis=1)
    bo = jnp.zeros((8, H), f32).at[0, :A].set(b_out)
    out = _head(h, seq3, wo, bo)
    return out[:, 0, 0]


# speedup vs baseline: 11.2825x; 1.0714x over previous
import functools

import jax
import jax.numpy as jnp
from jax import lax
from jax.experimental import pallas as pl
from jax.experimental.pallas import tpu as pltpu
from jax.experimental.pallas import tpu_sc as plsc

B, L, K, H, A, NRBF, NL = 4, 512, 48, 128, 21, 16, 3
N = B * L
E_TOT = N * K
NW = 32
ROWS_W = E_TOT // NW
CHUNK = 128
NCH = ROWS_W // CHUNK
NT = 128

_SIGMA = (22.0 - 2.0) / NRBF


def _ln(x):
    mu = x.mean(-1, keepdims=True)
    var = ((x - mu) ** 2).mean(-1, keepdims=True)
    return (x - mu) / jnp.sqrt(var + 1e-5)


RT = 128


def _centers_row():
    return 2.0 + lax.broadcasted_iota(jnp.int32, (1, NRBF), 1).astype(
        jnp.float32) * (20.0 / (NRBF - 1))


def _stage_a_body(ca_s_ref, ca_t_ref, gq_ref, vqc_ref, gidx_ref, dnb_ref,
                  isig_ref):
    b = pl.program_id(0)
    cs = ca_s_ref[0]
    ct = ca_t_ref[0]
    dx = cs[:, 0:1] - ct[0:1, :]
    dy = cs[:, 1:2] - ct[1:2, :]
    dz = cs[:, 2:3] - ct[2:3, :]
    d = jnp.sqrt(dx * dx + dy * dy + dz * dz + 1e-6)
    iot = lax.broadcasted_iota(jnp.int32, (RT, L), 1)
    centers = _centers_row()
    vq_col = vqc_ref[0:1, :].reshape(NRBF, 1)
    cq = vqc_ref[1, 0]
    key = (lax.bitcast_convert_type(d, jnp.int32) & jnp.int32(~511)) | iot
    dcols, icols = [], []
    for _ in range(K):
        kmin = jnp.min(key, axis=1, keepdims=True)
        im = kmin & 511
        dcols.append(lax.bitcast_convert_type(kmin - im, jnp.float32))
        icols.append(im)
        key = jnp.where(iot == im, jnp.int32(0x7FFFFFFF), key)
    gidx_ref[0] = jnp.concatenate(icols, axis=1) + b * L
    dnb_ref[0] = jnp.concatenate(dcols, axis=1)
    dtall = jnp.concatenate(dcols, axis=0)
    r = jnp.exp(-(((dtall - centers) / _SIGMA) ** 2))
    t = jnp.dot(r, gq_ref[...], preferred_element_type=jnp.float32)
    var = jnp.sum(t * r, axis=1, keepdims=True)
    var = var + 2.0 * jnp.dot(r, vq_col,
                              preferred_element_type=jnp.float32) + cq
    istall = 1.0 / jnp.sqrt(var + 1e-5)
    isig_ref[0] = jnp.concatenate(
        [istall[k * RT:(k + 1) * RT] for k in range(K)], axis=1)


def _stage_a(ca_s, ca_t, gq, vqc):
    return pl.pallas_call(
        _stage_a_body,
        grid=(B, L // RT),
        in_specs=[
            pl.BlockSpec((1, RT, 8), lambda b, t: (b, t, 0)),
            pl.BlockSpec((1, 8, L), lambda b, t: (b, 0, 0)),
            pl.BlockSpec((NRBF, NRBF), lambda b, t: (0, 0)),
            pl.BlockSpec((8, NRBF), lambda b, t: (0, 0)),
        ],
        out_specs=[
            pl.BlockSpec((1, RT, K), lambda b, t: (b, t, 0)),
            pl.BlockSpec((1, RT, K), lambda b, t: (b, t, 0)),
            pl.BlockSpec((1, RT, K), lambda b, t: (b, t, 0)),
        ],
        out_shape=[
            jax.ShapeDtypeStruct((B, L, K), jnp.int32),
            jax.ShapeDtypeStruct((B, L, K), jnp.float32),
            jax.ShapeDtypeStruct((B, L, K), jnp.float32),
        ],
    )(ca_s, ca_t, gq, vqc)


def _prep_body(seq_ref, t_ref, out_ref):
    sc = seq_ref[0]
    iot = lax.broadcasted_iota(jnp.int32, (L, 32), 1)
    oh = (sc == iot).astype(jnp.float32)
    out_ref[0, 0] = jnp.dot(oh, t_ref[0],
                            preferred_element_type=jnp.float32)


def _prep(seq3, tpad):
    return pl.pallas_call(
        _prep_body,
        grid=(NL, B),
        in_specs=[
            pl.BlockSpec((1, L, 1), lambda l, b: (b, 0, 0)),
            pl.BlockSpec((1, 32, H), lambda l, b: (l, 0, 0)),
        ],
        out_specs=pl.BlockSpec((1, 1, L, H), lambda l, b: (l, b, 0, 0)),
        out_shape=jax.ShapeDtypeStruct((NL, B, L, H), jnp.float32),
    )(seq3, tpad)


def _layer_body(has_g, has_hs, has_p, *refs):
    i = 0
    if has_g:
        h_ref, g_ref = refs[0], refs[1]
        i = 2
    dnb_ref, isig_ref, m_ref, cb_ref, w1a_ref, w2_ref = refs[i:i + 6]
    i += 6
    if has_p:
        w1bn_ref = refs[i]
        i += 1
    if has_hs:
        hsn_ref = refs[i]
        i += 1
    hn_ref = refs[i]
    if has_p:
        p_ref = refs[i + 1]

    dnb = dnb_ref[...]
    isg = isig_ref[...]
    centers = _centers_row()
    rfeat = jnp.concatenate(
        [jnp.exp(-(((dnb[:, k:k + 1] - centers) / _SIGMA) ** 2))
         for k in range(K)], axis=0)
    stall = jnp.concatenate(
        [isg[:, k:k + 1] for k in range(K)], axis=0)
    rtall = jnp.dot(rfeat, m_ref[...],
                    preferred_element_type=jnp.float32)
    mfull = (rtall + cb_ref[0:1, :]) * stall
    if has_g:
        h = h_ref[...]
        hi = jnp.dot(h, w1a_ref[...],
                     preferred_element_type=jnp.float32) + cb_ref[1:2, :]
        g2 = g_ref[...].reshape(K * NT, H)
        mfull = mfull + g2 + jnp.concatenate([hi] * K, axis=0)
    else:
        mfull = mfull + cb_ref[1:2, :]
    gf = jax.nn.gelu(mfull)
    s = gf[0:NT]
    for k in range(1, K):
        s = s + gf[k * NT:(k + 1) * NT]
    s = s / jnp.float32(K)
    u = jnp.dot(s, w2_ref[...], preferred_element_type=jnp.float32)
    u = u + cb_ref[2:3, :]
    hn = _ln(h + u) if has_g else _ln(u)
    hn_ref[...] = hn
    if has_p:
        p = jnp.dot(hn, w1bn_ref[...], preferred_element_type=jnp.float32)
        if has_hs:
            p = p + hsn_ref[...]
        p_ref[...] = p


TPB = L // NT


def _layer(h, g, dnb, isig, m, cb, w1a, w2, w1bn, hsn):
    has_g = g is not None
    has_p = w1bn is not None
    has_hs = hsn is not None
    steps = N // NT
    node2 = pl.BlockSpec((NT, H), lambda i: (i, 0))
    const = lambda shape: pl.BlockSpec(shape, lambda i: (0, 0))
    in_specs, args = [], []
    if has_g:
        in_specs += [node2,
                     pl.BlockSpec((1, K, NT, H),
                                  lambda i: (i // TPB, 0, i % TPB, 0))]
        args += [h, g]
    in_specs += [pl.BlockSpec((NT, K), lambda i: (i, 0)),
                 pl.BlockSpec((NT, K), lambda i: (i, 0)),
                 const((NRBF, H)), const((8, H)), const((H, H)),
                 const((H, H))]
    args += [dnb, isig, m, cb, w1a, w2]
    if has_p:
        in_specs.append(const((H, H)))
        args.append(w1bn)
    if has_hs:
        in_specs.append(node2)
        args.append(hsn)
    out_specs = [node2]
    out_shape = [jax.ShapeDtypeStruct((N, H), jnp.float32)]
    if has_p:
        out_specs.append(node2)
        out_shape.append(jax.ShapeDtypeStruct((N, H), jnp.float32))
    outs = pl.pallas_call(
        functools.partial(_layer_body, has_g, has_hs, has_p),
        grid=(steps,),
        in_specs=in_specs,
        out_specs=out_specs,
        out_shape=out_shape,
    )(*args)
    return outs if has_p else (outs[0], None)


def _head_body(h_ref, seq_ref, wo_ref, bo_ref, out_ref):
    logits = jnp.dot(h_ref[...], wo_ref[...],
                     preferred_element_type=jnp.float32) + bo_ref[0:1, :]
    lane = lax.broadcasted_iota(jnp.int32, (L, H), 1)
    valid = lane < A
    neg = jnp.float32(-1e30)
    mx = jnp.max(jnp.where(valid, logits, neg), axis=1, keepdims=True)
    ex = jnp.where(valid, jnp.exp(logits - mx), 0.0)
    lse = jnp.log(jnp.sum(ex, axis=1, keepdims=True)) + mx
    sc = seq_ref[0]
    sel = jnp.sum(jnp.where(lane == sc, logits, 0.0), axis=1,
                  keepdims=True)
    val = jnp.sum(sel - lse) / jnp.float32(L)
    out_ref[...] = jnp.full((1, 1, H), val, jnp.float32)


def _head(h, seq3, wo, bo):
    return pl.pallas_call(
        _head_body,
        grid=(B,),
        in_specs=[
            pl.BlockSpec((L, H), lambda b: (b, 0)),
            pl.BlockSpec((1, L, 1), lambda b: (b, 0, 0)),
            pl.BlockSpec((H, H), lambda b: (0, 0)),
            pl.BlockSpec((8, H), lambda b: (0, 0)),
        ],
        out_specs=pl.BlockSpec((1, 1, H), lambda b: (b, 0, 0)),
        out_shape=jax.ShapeDtypeStruct((B, 1, H), jnp.float32),
    )(h, seq3, wo, bo)


def _sc_gather(table, gidx3):
    mesh = plsc.VectorSubcoreMesh(core_axis_name="c", subcore_axis_name="s")

    @functools.partial(
        pl.kernel, mesh=mesh,
        out_type=jax.ShapeDtypeStruct((NW, NCH, CHUNK, H), jnp.float32),
        scratch_types=[
            pltpu.VMEM((NCH, CHUNK), jnp.int32),
            pltpu.VMEM((CHUNK, H), jnp.float32),
            pltpu.VMEM((CHUNK, H), jnp.float32),
            pltpu.SemaphoreType.DMA,
            pltpu.SemaphoreType.DMA,
        ],
    )
    def k(table_hbm, idx_hbm, out_hbm, idx_v, rows0, rows1, sem0, sem1):
        w = lax.axis_index("s") * 2 + lax.axis_index("c")
        pltpu.sync_copy(idx_hbm.at[w], idx_v)
        bufs = (rows0, rows1)
        sems = (sem0, sem1)
        for t in range(2):
            pltpu.make_async_copy(table_hbm.at[idx_v.at[t]], bufs[t],
                                  sems[t]).start()

        def body(i, carry):
            j0 = 2 * i
            for t in range(2):
                j = j0 + t
                pltpu.make_async_copy(table_hbm.at[idx_v.at[j]], bufs[t],
                                      sems[t]).wait()
                pltpu.sync_copy(bufs[t], out_hbm.at[w, j])

                @pl.when(j + 2 < NCH)
                def _():
                    pltpu.make_async_copy(table_hbm.at[idx_v.at[j + 2]],
                                          bufs[t], sems[t]).start()
            return carry

        lax.fori_loop(0, NCH // 2, body, 0)

    return k(table, gidx3)


def _gather_edges(p, gidx3):
    out = _sc_gather(p, gidx3)
    return out.reshape(B, K, L, H)


def kernel(seq_samples, structure, s_emb, W_e, b_e, enc_W1, enc_b1, enc_W2,
           enc_b2, dec_W1, dec_b1, dec_W2, dec_b2, W_out, b_out):
    f32 = jnp.float32
    seq3 = seq_samples.astype(jnp.int32)[..., None]
    ca = structure[:, :, 1, :]
    ca_s = jnp.concatenate([ca, jnp.zeros((B, L, 5), f32)], axis=2)
    ca_t = jnp.swapaxes(ca_s, 1, 2)

    wbar = W_e.mean(axis=1, keepdims=True)
    wc = W_e - wbar
    bc = b_e - b_e.mean()
    gq = jnp.dot(wc, wc.T) / H
    vq = jnp.dot(wc, bc) / H
    vqc = jnp.zeros((8, NRBF), f32).at[0].set(vq).at[1, 0].set(
        jnp.dot(bc, bc) / H)

    gidx, dnb, isig = _stage_a(ca_s, ca_t, gq, vqc)
    gidx3 = gidx.transpose(0, 2, 1).reshape(NW, NCH, CHUNK)
    dnb = dnb.reshape(N, K)
    isig = isig.reshape(N, K)

    eW1a = [enc_W1[l][:H] for l in range(NL)]
    eW1b = [enc_W1[l][H:2 * H] for l in range(NL)]
    eM = [jnp.dot(wc, enc_W1[l][2 * H:]) for l in range(NL)]
    ec = [jnp.dot(bc, enc_W1[l][2 * H:]) for l in range(NL)]
    dW1a = [dec_W1[l][:H] for l in range(NL)]
    dW1b = [dec_W1[l][H:2 * H] for l in range(NL)]
    dT = jnp.stack([jnp.dot(s_emb, dec_W1[l][2 * H:3 * H])
                    for l in range(NL)])
    dM = [jnp.dot(wc, dec_W1[l][3 * H:]) for l in range(NL)]
    dc = [jnp.dot(bc, dec_W1[l][3 * H:]) for l in range(NL)]

    def cbpack(c, b1, b2):
        z = jnp.zeros((8, H), f32)
        return z.at[0].set(c).at[1].set(b1).at[2].set(b2)

    ecb = [cbpack(ec[l], enc_b1[l], enc_b2[l]) for l in range(NL)]
    dcb = [cbpack(dc[l], dec_b1[l], dec_b2[l]) for l in range(NL)]

    tpad = jnp.concatenate([dT, jnp.zeros((NL, 32 - A, H), f32)], axis=1)
    hs3 = _prep(seq3, tpad).reshape(NL, N, H)

    h, p = _layer(None, None, dnb, isig, eM[0], ecb[0], eW1a[0], enc_W2[0],
                  eW1b[1], None)
    g = _gather_edges(p, gidx3)
    h, p = _layer(h, g, dnb, isig, eM[1], ecb[1], eW1a[1], enc_W2[1],
                  eW1b[2], None)
    g = _gather_edges(p, gidx3)
    h, p = _layer(h, g, dnb, isig, eM[2], ecb[2], eW1a[2], enc_W2[2],
                  dW1b[0], hs3[0])
    g = _gather_edges(p, gidx3)
    h, p = _layer(h, g, dnb, isig, dM[0], dcb[0], dW1a[0], dec_W2[0],
                  dW1b[1], hs3[1])
    g = _gather_edges(p, gidx3)
    h, p = _layer(h, g, dnb, isig, dM[1], dcb[1], dW1a[1], dec_W2[1],
                  dW1b[2], hs3[2])
    g = _gather_edges(p, gidx3)
    h, _ = _layer(h, g, dnb, isig, dM[2], dcb[2], dW1a[2], dec_W2[2],
                  None, None)

    wo = jnp.concatenate([W_out, jnp.zeros((H, H - A), f32)], axis=1)
    bo = jnp.zeros((8, H), f32).at[0, :A].set(b_out)
    out = _head(h, seq3, wo, bo)
    return out[:, 0, 0]


# speedup vs baseline: 11.4660x; 1.0163x over previous
import functools

import jax
import jax.numpy as jnp
from jax import lax
from jax.experimental import pallas as pl
from jax.experimental.pallas import tpu as pltpu
from jax.experimental.pallas import tpu_sc as plsc

B, L, K, H, A, NRBF, NL = 4, 512, 48, 128, 21, 16, 3
N = B * L
E_TOT = N * K
NW = 32
ROWS_W = E_TOT // NW
CHUNK = 128
NCH = ROWS_W // CHUNK
NG = 3
NGRP = NCH // NG
NT = 256

_SIGMA = (22.0 - 2.0) / NRBF


def _ln(x):
    mu = x.mean(-1, keepdims=True)
    var = ((x - mu) ** 2).mean(-1, keepdims=True)
    return (x - mu) / jnp.sqrt(var + 1e-5)


RT = 128


def _centers_row():
    return 2.0 + lax.broadcasted_iota(jnp.int32, (1, NRBF), 1).astype(
        jnp.float32) * (20.0 / (NRBF - 1))


def _stage_a_body(ca_s_ref, ca_t_ref, gq_ref, vqc_ref, gidx_ref, dnb_ref,
                  isig_ref):
    b = pl.program_id(0)
    cs = ca_s_ref[0]
    ct = ca_t_ref[0]
    dx = cs[:, 0:1] - ct[0:1, :]
    dy = cs[:, 1:2] - ct[1:2, :]
    dz = cs[:, 2:3] - ct[2:3, :]
    d = jnp.sqrt(dx * dx + dy * dy + dz * dz + 1e-6)
    iot = lax.broadcasted_iota(jnp.int32, (RT, L), 1)
    centers = _centers_row()
    vq_col = vqc_ref[0:1, :].reshape(NRBF, 1)
    cq = vqc_ref[1, 0]
    key = (lax.bitcast_convert_type(d, jnp.int32) & jnp.int32(~511)) | iot
    dcols, icols = [], []
    for _ in range(K):
        kmin = jnp.min(key, axis=1, keepdims=True)
        im = kmin & 511
        dcols.append(lax.bitcast_convert_type(kmin - im, jnp.float32))
        icols.append(im)
        key = jnp.where(iot == im, jnp.int32(0x7FFFFFFF), key)
    gidx_ref[0] = jnp.concatenate(icols, axis=1) + b * L
    dnb_ref[0] = jnp.concatenate(dcols, axis=1)
    dtall = jnp.concatenate(dcols, axis=0)
    r = jnp.exp(-(((dtall - centers) / _SIGMA) ** 2))
    t = jnp.dot(r, gq_ref[...], preferred_element_type=jnp.float32)
    var = jnp.sum(t * r, axis=1, keepdims=True)
    var = var + 2.0 * jnp.dot(r, vq_col,
                              preferred_element_type=jnp.float32) + cq
    istall = 1.0 / jnp.sqrt(var + 1e-5)
    isig_ref[0] = jnp.concatenate(
        [istall[k * RT:(k + 1) * RT] for k in range(K)], axis=1)


def _stage_a(ca_s, ca_t, gq, vqc):
    return pl.pallas_call(
        _stage_a_body,
        grid=(B, L // RT),
        in_specs=[
            pl.BlockSpec((1, RT, 8), lambda b, t: (b, t, 0)),
            pl.BlockSpec((1, 8, L), lambda b, t: (b, 0, 0)),
            pl.BlockSpec((NRBF, NRBF), lambda b, t: (0, 0)),
            pl.BlockSpec((8, NRBF), lambda b, t: (0, 0)),
        ],
        out_specs=[
            pl.BlockSpec((1, RT, K), lambda b, t: (b, t, 0)),
            pl.BlockSpec((1, RT, K), lambda b, t: (b, t, 0)),
            pl.BlockSpec((1, RT, K), lambda b, t: (b, t, 0)),
        ],
        out_shape=[
            jax.ShapeDtypeStruct((B, L, K), jnp.int32),
            jax.ShapeDtypeStruct((B, L, K), jnp.float32),
            jax.ShapeDtypeStruct((B, L, K), jnp.float32),
        ],
    )(ca_s, ca_t, gq, vqc)


def _prep_body(seq_ref, t_ref, out_ref):
    sc = seq_ref[0]
    iot = lax.broadcasted_iota(jnp.int32, (L, 32), 1)
    oh = (sc == iot).astype(jnp.float32)
    out_ref[0, 0] = jnp.dot(oh, t_ref[0],
                            preferred_element_type=jnp.float32)


def _prep(seq3, tpad):
    return pl.pallas_call(
        _prep_body,
        grid=(NL, B),
        in_specs=[
            pl.BlockSpec((1, L, 1), lambda l, b: (b, 0, 0)),
            pl.BlockSpec((1, 32, H), lambda l, b: (l, 0, 0)),
        ],
        out_specs=pl.BlockSpec((1, 1, L, H), lambda l, b: (l, b, 0, 0)),
        out_shape=jax.ShapeDtypeStruct((NL, B, L, H), jnp.float32),
    )(seq3, tpad)


def _layer_body(has_g, has_hs, has_p, *refs):
    i = 0
    if has_g:
        h_ref, g_ref = refs[0], refs[1]
        i = 2
    dnb_ref, isig_ref, m_ref, cb_ref, w1a_ref, w2_ref = refs[i:i + 6]
    i += 6
    if has_p:
        w1bn_ref = refs[i]
        i += 1
    if has_hs:
        hsn_ref = refs[i]
        i += 1
    hn_ref = refs[i]
    if has_p:
        p_ref = refs[i + 1]

    dnb = dnb_ref[...]
    isg = isig_ref[...]
    centers = _centers_row()
    rfeat = jnp.concatenate(
        [jnp.exp(-(((dnb[:, k:k + 1] - centers) / _SIGMA) ** 2))
         for k in range(K)], axis=0)
    stall = jnp.concatenate(
        [isg[:, k:k + 1] for k in range(K)], axis=0)
    rtall = jnp.dot(rfeat, m_ref[...],
                    preferred_element_type=jnp.float32)
    mfull = (rtall + cb_ref[0:1, :]) * stall
    if has_g:
        h = h_ref[...]
        hi = jnp.dot(h, w1a_ref[...],
                     preferred_element_type=jnp.float32) + cb_ref[1:2, :]
        g2 = g_ref[...].reshape(K * NT, H)
        mfull = mfull + g2 + jnp.concatenate([hi] * K, axis=0)
    else:
        mfull = mfull + cb_ref[1:2, :]
    gf = jax.nn.gelu(mfull)
    s = gf[0:NT]
    for k in range(1, K):
        s = s + gf[k * NT:(k + 1) * NT]
    s = s / jnp.float32(K)
    u = jnp.dot(s, w2_ref[...], preferred_element_type=jnp.float32)
    u = u + cb_ref[2:3, :]
    hn = _ln(h + u) if has_g else _ln(u)
    hn_ref[...] = hn
    if has_p:
        p = jnp.dot(hn, w1bn_ref[...], preferred_element_type=jnp.float32)
        if has_hs:
            p = p + hsn_ref[...]
        p_ref[...] = p


TPB = L // NT


def _layer(h, g, dnb, isig, m, cb, w1a, w2, w1bn, hsn):
    has_g = g is not None
    has_p = w1bn is not None
    has_hs = hsn is not None
    steps = N // NT
    node2 = pl.BlockSpec((NT, H), lambda i: (i, 0))
    const = lambda shape: pl.BlockSpec(shape, lambda i: (0, 0))
    in_specs, args = [], []
    if has_g:
        in_specs += [node2,
                     pl.BlockSpec((1, K, NT, H),
                                  lambda i: (i // TPB, 0, i % TPB, 0))]
        args += [h, g]
    in_specs += [pl.BlockSpec((NT, K), lambda i: (i, 0)),
                 pl.BlockSpec((NT, K), lambda i: (i, 0)),
                 const((NRBF, H)), const((8, H)), const((H, H)),
                 const((H, H))]
    args += [dnb, isig, m, cb, w1a, w2]
    if has_p:
        in_specs.append(const((H, H)))
        args.append(w1bn)
    if has_hs:
        in_specs.append(node2)
        args.append(hsn)
    out_specs = [node2]
    out_shape = [jax.ShapeDtypeStruct((N, H), jnp.float32)]
    if has_p:
        out_specs.append(node2)
        out_shape.append(jax.ShapeDtypeStruct((N, H), jnp.float32))
    outs = pl.pallas_call(
        functools.partial(_layer_body, has_g, has_hs, has_p),
        grid=(steps,),
        in_specs=in_specs,
        out_specs=out_specs,
        out_shape=out_shape,
    )(*args)
    return outs if has_p else (outs[0], None)


def _head_body(h_ref, seq_ref, wo_ref, bo_ref, out_ref):
    logits = jnp.dot(h_ref[...], wo_ref[...],
                     preferred_element_type=jnp.float32) + bo_ref[0:1, :]
    lane = lax.broadcasted_iota(jnp.int32, (L, H), 1)
    valid = lane < A
    neg = jnp.float32(-1e30)
    mx = jnp.max(jnp.where(valid, logits, neg), axis=1, keepdims=True)
    ex = jnp.where(valid, jnp.exp(logits - mx), 0.0)
    lse = jnp.log(jnp.sum(ex, axis=1, keepdims=True)) + mx
    sc = seq_ref[0]
    sel = jnp.sum(jnp.where(lane == sc, logits, 0.0), axis=1,
                  keepdims=True)
    val = jnp.sum(sel - lse) / jnp.float32(L)
    out_ref[...] = jnp.full((1, 1, H), val, jnp.float32)


def _head(h, seq3, wo, bo):
    return pl.pallas_call(
        _head_body,
        grid=(B,),
        in_specs=[
            pl.BlockSpec((L, H), lambda b: (b, 0)),
            pl.BlockSpec((1, L, 1), lambda b: (b, 0, 0)),
            pl.BlockSpec((H, H), lambda b: (0, 0)),
            pl.BlockSpec((8, H), lambda b: (0, 0)),
        ],
        out_specs=pl.BlockSpec((1, 1, H), lambda b: (b, 0, 0)),
        out_shape=jax.ShapeDtypeStruct((B, 1, H), jnp.float32),
    )(h, seq3, wo, bo)


def _sc_gather(table, gidx3):
    mesh = plsc.VectorSubcoreMesh(core_axis_name="c", subcore_axis_name="s")

    @functools.partial(
        pl.kernel, mesh=mesh,
        out_type=jax.ShapeDtypeStruct((NW, NCH, CHUNK, H), jnp.float32),
        scratch_types=[
            pltpu.VMEM((NCH, CHUNK), jnp.int32),
            pltpu.VMEM((CHUNK, H), jnp.float32),
            pltpu.VMEM((CHUNK, H), jnp.float32),
            pltpu.SemaphoreType.DMA,
            pltpu.SemaphoreType.DMA,
        ],
    )
    def k(table_hbm, idx_hbm, out_hbm, idx_v, rows0, rows1, sem0, sem1):
        w = lax.axis_index("s") * 2 + lax.axis_index("c")
        pltpu.sync_copy(idx_hbm.at[w], idx_v)
        bufs = (rows0, rows1)
        sems = (sem0, sem1)
        for t in range(2):
            pltpu.make_async_copy(table_hbm.at[idx_v.at[t]], bufs[t],
                                  sems[t]).start()

        def body(i, carry):
            j0 = 2 * i
            for t in range(2):
                j = j0 + t
                pltpu.make_async_copy(table_hbm.at[idx_v.at[j]], bufs[t],
                                      sems[t]).wait()
                pltpu.sync_copy(bufs[t], out_hbm.at[w, j])

                @pl.when(j + 2 < NCH)
                def _():
                    pltpu.make_async_copy(table_hbm.at[idx_v.at[j + 2]],
                                          bufs[t], sems[t]).start()
            return carry

        lax.fori_loop(0, NCH // 2, body, 0)

    return k(table, gidx3)


def _gather_edges(p, gidx3):
    out = _sc_gather(p, gidx3)
    return out.reshape(B, K, L, H)


def kernel(seq_samples, structure, s_emb, W_e, b_e, enc_W1, enc_b1, enc_W2,
           enc_b2, dec_W1, dec_b1, dec_W2, dec_b2, W_out, b_out):
    f32 = jnp.float32
    seq3 = seq_samples.astype(jnp.int32)[..., None]
    ca = structure[:, :, 1, :]
    ca_s = jnp.concatenate([ca, jnp.zeros((B, L, 5), f32)], axis=2)
    ca_t = jnp.swapaxes(ca_s, 1, 2)

    wbar = W_e.mean(axis=1, keepdims=True)
    wc = W_e - wbar
    bc = b_e - b_e.mean()
    gq = jnp.dot(wc, wc.T) / H
    vq = jnp.dot(wc, bc) / H
    vqc = jnp.zeros((8, NRBF), f32).at[0].set(vq).at[1, 0].set(
        jnp.dot(bc, bc) / H)

    gidx, dnb, isig = _stage_a(ca_s, ca_t, gq, vqc)
    gidx3 = gidx.transpose(0, 2, 1).reshape(NW, NCH, CHUNK)
    dnb = dnb.reshape(N, K)
    isig = isig.reshape(N, K)

    eW1a = [enc_W1[l][:H] for l in range(NL)]
    eW1b = [enc_W1[l][H:2 * H] for l in range(NL)]
    eM = [jnp.dot(wc, enc_W1[l][2 * H:]) for l in range(NL)]
    ec = [jnp.dot(bc, enc_W1[l][2 * H:]) for l in range(NL)]
    dW1a = [dec_W1[l][:H] for l in range(NL)]
    dW1b = [dec_W1[l][H:2 * H] for l in range(NL)]
    dT = jnp.stack([jnp.dot(s_emb, dec_W1[l][2 * H:3 * H])
                    for l in range(NL)])
    dM = [jnp.dot(wc, dec_W1[l][3 * H:]) for l in range(NL)]
    dc = [jnp.dot(bc, dec_W1[l][3 * H:]) for l in range(NL)]

    def cbpack(c, b1, b2):
        z = jnp.zeros((8, H), f32)
        return z.at[0].set(c).at[1].set(b1).at[2].set(b2)

    ecb = [cbpack(ec[l], enc_b1[l], enc_b2[l]) for l in range(NL)]
    dcb = [cbpack(dc[l], dec_b1[l], dec_b2[l]) for l in range(NL)]

    tpad = jnp.concatenate([dT, jnp.zeros((NL, 32 - A, H), f32)], axis=1)
    hs3 = _prep(seq3, tpad).reshape(NL, N, H)

    h, p = _layer(None, None, dnb, isig, eM[0], ecb[0], eW1a[0], enc_W2[0],
                  eW1b[1], None)
    g = _gather_edges(p, gidx3)
    h, p = _layer(h, g, dnb, isig, eM[1], ecb[1], eW1a[1], enc_W2[1],
                  eW1b[2], None)
    g = _gather_edges(p, gidx3)
    h, p = _layer(h, g, dnb, isig, eM[2], ecb[2], eW1a[2], enc_W2[2],
                  dW1b[0], hs3[0])
    g = _gather_edges(p, gidx3)
    h, p = _layer(h, g, dnb, isig, dM[0], dcb[0], dW1a[0], dec_W2[0],
                  dW1b[1], hs3[1])
    g = _gather_edges(p, gidx3)
    h, p = _layer(h, g, dnb, isig, dM[1], dcb[1], dW1a[1], dec_W2[1],
                  dW1b[2], hs3[2])
    g = _gather_edges(p, gidx3)
    h, _ = _layer(h, g, dnb, isig, dM[2], dcb[2], dW1a[2], dec_W2[2],
                  None, None)

    wo = jnp.concatenate([W_out, jnp.zeros((H, H - A), f32)], axis=1)
    bo = jnp.zeros((8, H), f32).at[0, :A].set(b_out)
    out = _head(h, seq3, wo, bo)
    return out[:, 0, 0]


# speedup vs baseline: 12.3986x; 1.0813x over previous
import functools

import jax
import jax.numpy as jnp
from jax import lax
from jax.experimental import pallas as pl
from jax.experimental.pallas import tpu as pltpu
from jax.experimental.pallas import tpu_sc as plsc

B, L, K, H, A, NRBF, NL = 4, 512, 48, 128, 21, 16, 3
N = B * L
E_TOT = N * K
NW = 32
ROWS_W = E_TOT // NW
CHUNK = 128
NCH = ROWS_W // CHUNK
NG = 3
NGRP = NCH // NG
NT = 256

_SIGMA = (22.0 - 2.0) / NRBF


def _ln(x):
    mu = x.mean(-1, keepdims=True)
    var = ((x - mu) ** 2).mean(-1, keepdims=True)
    return (x - mu) / jnp.sqrt(var + 1e-5)


RT = 256


def _centers_row():
    return 2.0 + lax.broadcasted_iota(jnp.int32, (1, NRBF), 1).astype(
        jnp.float32) * (20.0 / (NRBF - 1))


def _stage_a_body(ca_s_ref, ca_t_ref, gq_ref, vqc_ref, gidx_ref, dnb_ref,
                  isig_ref):
    b = pl.program_id(0)
    cs = ca_s_ref[0]
    ct = ca_t_ref[0]
    dx = cs[:, 0:1] - ct[0:1, :]
    dy = cs[:, 1:2] - ct[1:2, :]
    dz = cs[:, 2:3] - ct[2:3, :]
    d = jnp.sqrt(dx * dx + dy * dy + dz * dz + 1e-6)
    iot = lax.broadcasted_iota(jnp.int32, (RT, L), 1)
    centers = _centers_row()
    vq_col = vqc_ref[0:1, :].reshape(NRBF, 1)
    cq = vqc_ref[1, 0]
    key = (lax.bitcast_convert_type(d, jnp.int32) & jnp.int32(~511)) | iot
    dcols, icols = [], []
    for _ in range(K):
        kmin = jnp.min(key, axis=1, keepdims=True)
        im = kmin & 511
        dcols.append(lax.bitcast_convert_type(kmin - im, jnp.float32))
        icols.append(im)
        key = jnp.where(iot == im, jnp.int32(0x7FFFFFFF), key)
    gidx_ref[0] = jnp.concatenate(icols, axis=1) + b * L
    dnb_ref[0] = jnp.concatenate(dcols, axis=1)
    dtall = jnp.concatenate(dcols, axis=0)
    r = jnp.exp(-(((dtall - centers) / _SIGMA) ** 2))
    t = jnp.dot(r, gq_ref[...], preferred_element_type=jnp.float32)
    var = jnp.sum(t * r, axis=1, keepdims=True)
    var = var + 2.0 * jnp.dot(r, vq_col,
                              preferred_element_type=jnp.float32) + cq
    istall = 1.0 / jnp.sqrt(var + 1e-5)
    isig_ref[0] = jnp.concatenate(
        [istall[k * RT:(k + 1) * RT] for k in range(K)], axis=1)


def _stage_a(ca_s, ca_t, gq, vqc):
    return pl.pallas_call(
        _stage_a_body,
        grid=(B, L // RT),
        in_specs=[
            pl.BlockSpec((1, RT, 8), lambda b, t: (b, t, 0)),
            pl.BlockSpec((1, 8, L), lambda b, t: (b, 0, 0)),
            pl.BlockSpec((NRBF, NRBF), lambda b, t: (0, 0)),
            pl.BlockSpec((8, NRBF), lambda b, t: (0, 0)),
        ],
        out_specs=[
            pl.BlockSpec((1, RT, K), lambda b, t: (b, t, 0)),
            pl.BlockSpec((1, RT, K), lambda b, t: (b, t, 0)),
            pl.BlockSpec((1, RT, K), lambda b, t: (b, t, 0)),
        ],
        out_shape=[
            jax.ShapeDtypeStruct((B, L, K), jnp.int32),
            jax.ShapeDtypeStruct((B, L, K), jnp.float32),
            jax.ShapeDtypeStruct((B, L, K), jnp.float32),
        ],
    )(ca_s, ca_t, gq, vqc)


def _prep_body(seq_ref, t_ref, out_ref):
    sc = seq_ref[0]
    iot = lax.broadcasted_iota(jnp.int32, (L, 32), 1)
    oh = (sc == iot).astype(jnp.float32)
    out_ref[0, 0] = jnp.dot(oh, t_ref[0],
                            preferred_element_type=jnp.float32)


def _prep(seq3, tpad):
    return pl.pallas_call(
        _prep_body,
        grid=(NL, B),
        in_specs=[
            pl.BlockSpec((1, L, 1), lambda l, b: (b, 0, 0)),
            pl.BlockSpec((1, 32, H), lambda l, b: (l, 0, 0)),
        ],
        out_specs=pl.BlockSpec((1, 1, L, H), lambda l, b: (l, b, 0, 0)),
        out_shape=jax.ShapeDtypeStruct((NL, B, L, H), jnp.float32),
    )(seq3, tpad)


def _layer_body(has_g, has_hs, has_p, *refs):
    i = 0
    if has_g:
        h_ref, g_ref = refs[0], refs[1]
        i = 2
    dnb_ref, isig_ref, m_ref, cb_ref, w1a_ref, w2_ref = refs[i:i + 6]
    i += 6
    if has_p:
        w1bn_ref = refs[i]
        i += 1
    if has_hs:
        hsn_ref = refs[i]
        i += 1
    hn_ref = refs[i]
    if has_p:
        p_ref = refs[i + 1]

    dnb = dnb_ref[...]
    isg = isig_ref[...]
    centers = _centers_row()
    rfeat = jnp.concatenate(
        [jnp.exp(-(((dnb[:, k:k + 1] - centers) / _SIGMA) ** 2))
         for k in range(K)], axis=0)
    stall = jnp.concatenate(
        [isg[:, k:k + 1] for k in range(K)], axis=0)
    rtall = jnp.dot(rfeat, m_ref[...],
                    preferred_element_type=jnp.float32)
    mfull = (rtall + cb_ref[0:1, :]) * stall
    if has_g:
        h = h_ref[...]
        hi = jnp.dot(h, w1a_ref[...],
                     preferred_element_type=jnp.float32) + cb_ref[1:2, :]
        g2 = g_ref[...].reshape(K * NT, H)
        mfull = mfull + g2 + jnp.concatenate([hi] * K, axis=0)
    else:
        mfull = mfull + cb_ref[1:2, :]
    gf = jax.nn.gelu(mfull)
    s = gf[0:NT]
    for k in range(1, K):
        s = s + gf[k * NT:(k + 1) * NT]
    s = s / jnp.float32(K)
    u = jnp.dot(s, w2_ref[...], preferred_element_type=jnp.float32)
    u = u + cb_ref[2:3, :]
    hn = _ln(h + u) if has_g else _ln(u)
    hn_ref[...] = hn
    if has_p:
        p = jnp.dot(hn, w1bn_ref[...], preferred_element_type=jnp.float32)
        if has_hs:
            p = p + hsn_ref[...]
        p_ref[...] = p


TPB = L // NT


def _layer(h, g, dnb, isig, m, cb, w1a, w2, w1bn, hsn):
    has_g = g is not None
    has_p = w1bn is not None
    has_hs = hsn is not None
    steps = N // NT
    node2 = pl.BlockSpec((NT, H), lambda i: (i, 0))
    const = lambda shape: pl.BlockSpec(shape, lambda i: (0, 0))
    in_specs, args = [], []
    if has_g:
        in_specs += [node2,
                     pl.BlockSpec((1, K, NT, H),
                                  lambda i: (i // TPB, 0, i % TPB, 0))]
        args += [h, g]
    in_specs += [pl.BlockSpec((NT, K), lambda i: (i, 0)),
                 pl.BlockSpec((NT, K), lambda i: (i, 0)),
                 const((NRBF, H)), const((8, H)), const((H, H)),
                 const((H, H))]
    args += [dnb, isig, m, cb, w1a, w2]
    if has_p:
        in_specs.append(const((H, H)))
        args.append(w1bn)
    if has_hs:
        in_specs.append(node2)
        args.append(hsn)
    out_specs = [node2]
    out_shape = [jax.ShapeDtypeStruct((N, H), jnp.float32)]
    if has_p:
        out_specs.append(node2)
        out_shape.append(jax.ShapeDtypeStruct((N, H), jnp.float32))
    outs = pl.pallas_call(
        functools.partial(_layer_body, has_g, has_hs, has_p),
        grid=(steps,),
        in_specs=in_specs,
        out_specs=out_specs,
        out_shape=out_shape,
    )(*args)
    return outs if has_p else (outs[0], None)


def _head_body(h_ref, seq_ref, wo_ref, bo_ref, out_ref):
    logits = jnp.dot(h_ref[...], wo_ref[...],
                     preferred_element_type=jnp.float32) + bo_ref[0:1, :]
    lane = lax.broadcasted_iota(jnp.int32, (L, H), 1)
    valid = lane < A
    neg = jnp.float32(-1e30)
    mx = jnp.max(jnp.where(valid, logits, neg), axis=1, keepdims=True)
    ex = jnp.where(valid, jnp.exp(logits - mx), 0.0)
    lse = jnp.log(jnp.sum(ex, axis=1, keepdims=True)) + mx
    sc = seq_ref[0]
    sel = jnp.sum(jnp.where(lane == sc, logits, 0.0), axis=1,
                  keepdims=True)
    val = jnp.sum(sel - lse) / jnp.float32(L)
    out_ref[...] = jnp.full((1, 1, H), val, jnp.float32)


def _head(h, seq3, wo, bo):
    return pl.pallas_call(
        _head_body,
        grid=(B,),
        in_specs=[
            pl.BlockSpec((L, H), lambda b: (b, 0)),
            pl.BlockSpec((1, L, 1), lambda b: (b, 0, 0)),
            pl.BlockSpec((H, H), lambda b: (0, 0)),
            pl.BlockSpec((8, H), lambda b: (0, 0)),
        ],
        out_specs=pl.BlockSpec((1, 1, H), lambda b: (b, 0, 0)),
        out_shape=jax.ShapeDtypeStruct((B, 1, H), jnp.float32),
    )(h, seq3, wo, bo)


def _sc_gather(table, gidx3):
    mesh = plsc.VectorSubcoreMesh(core_axis_name="c", subcore_axis_name="s")

    @functools.partial(
        pl.kernel, mesh=mesh,
        out_type=jax.ShapeDtypeStruct((NW, NCH, CHUNK, H), jnp.float32),
        scratch_types=[
            pltpu.VMEM((NCH, CHUNK), jnp.int32),
            pltpu.VMEM((CHUNK, H), jnp.float32),
            pltpu.VMEM((CHUNK, H), jnp.float32),
            pltpu.SemaphoreType.DMA,
            pltpu.SemaphoreType.DMA,
        ],
    )
    def k(table_hbm, idx_hbm, out_hbm, idx_v, rows0, rows1, sem0, sem1):
        w = lax.axis_index("s") * 2 + lax.axis_index("c")
        pltpu.sync_copy(idx_hbm.at[w], idx_v)
        bufs = (rows0, rows1)
        sems = (sem0, sem1)
        for t in range(2):
            pltpu.make_async_copy(table_hbm.at[idx_v.at[t]], bufs[t],
                                  sems[t]).start()

        def body(i, carry):
            j0 = 2 * i
            for t in range(2):
                j = j0 + t
                pltpu.make_async_copy(table_hbm.at[idx_v.at[j]], bufs[t],
                                      sems[t]).wait()
                pltpu.sync_copy(bufs[t], out_hbm.at[w, j])

                @pl.when(j + 2 < NCH)
                def _():
                    pltpu.make_async_copy(table_hbm.at[idx_v.at[j + 2]],
                                          bufs[t], sems[t]).start()
            return carry

        lax.fori_loop(0, NCH // 2, body, 0)

    return k(table, gidx3)


def _gather_edges(p, gidx3):
    out = _sc_gather(p, gidx3)
    return out.reshape(B, K, L, H)


def kernel(seq_samples, structure, s_emb, W_e, b_e, enc_W1, enc_b1, enc_W2,
           enc_b2, dec_W1, dec_b1, dec_W2, dec_b2, W_out, b_out):
    f32 = jnp.float32
    seq3 = seq_samples.astype(jnp.int32)[..., None]
    ca = structure[:, :, 1, :]
    ca_s = jnp.concatenate([ca, jnp.zeros((B, L, 5), f32)], axis=2)
    ca_t = jnp.swapaxes(ca_s, 1, 2)

    wbar = W_e.mean(axis=1, keepdims=True)
    wc = W_e - wbar
    bc = b_e - b_e.mean()
    gq = jnp.dot(wc, wc.T) / H
    vq = jnp.dot(wc, bc) / H
    vqc = jnp.zeros((8, NRBF), f32).at[0].set(vq).at[1, 0].set(
        jnp.dot(bc, bc) / H)

    gidx, dnb, isig = _stage_a(ca_s, ca_t, gq, vqc)
    gidx3 = gidx.transpose(0, 2, 1).reshape(NW, NCH, CHUNK)
    dnb = dnb.reshape(N, K)
    isig = isig.reshape(N, K)

    eW1a = [enc_W1[l][:H] for l in range(NL)]
    eW1b = [enc_W1[l][H:2 * H] for l in range(NL)]
    eM = [jnp.dot(wc, enc_W1[l][2 * H:]) for l in range(NL)]
    ec = [jnp.dot(bc, enc_W1[l][2 * H:]) for l in range(NL)]
    dW1a = [dec_W1[l][:H] for l in range(NL)]
    dW1b = [dec_W1[l][H:2 * H] for l in range(NL)]
    dT = jnp.stack([jnp.dot(s_emb, dec_W1[l][2 * H:3 * H])
                    for l in range(NL)])
    dM = [jnp.dot(wc, dec_W1[l][3 * H:]) for l in range(NL)]
    dc = [jnp.dot(bc, dec_W1[l][3 * H:]) for l in range(NL)]

    def cbpack(c, b1, b2):
        z = jnp.zeros((8, H), f32)
        return z.at[0].set(c).at[1].set(b1).at[2].set(b2)

    ecb = [cbpack(ec[l], enc_b1[l], enc_b2[l]) for l in range(NL)]
    dcb = [cbpack(dc[l], dec_b1[l], dec_b2[l]) for l in range(NL)]

    tpad = jnp.concatenate([dT, jnp.zeros((NL, 32 - A, H), f32)], axis=1)
    hs3 = _prep(seq3, tpad).reshape(NL, N, H)

    h, p = _layer(None, None, dnb, isig, eM[0], ecb[0], eW1a[0], enc_W2[0],
                  eW1b[1], None)
    g = _gather_edges(p, gidx3)
    h, p = _layer(h, g, dnb, isig, eM[1], ecb[1], eW1a[1], enc_W2[1],
                  eW1b[2], None)
    g = _gather_edges(p, gidx3)
    h, p = _layer(h, g, dnb, isig, eM[2], ecb[2], eW1a[2], enc_W2[2],
                  dW1b[0], hs3[0])
    g = _gather_edges(p, gidx3)
    h, p = _layer(h, g, dnb, isig, dM[0], dcb[0], dW1a[0], dec_W2[0],
                  dW1b[1], hs3[1])
    g = _gather_edges(p, gidx3)
    h, p = _layer(h, g, dnb, isig, dM[1], dcb[1], dW1a[1], dec_W2[1],
                  dW1b[2], hs3[2])
    g = _gather_edges(p, gidx3)
    h, _ = _layer(h, g, dnb, isig, dM[2], dcb[2], dW1a[2], dec_W2[2],
                  None, None)

    wo = jnp.concatenate([W_out, jnp.zeros((H, H - A), f32)], axis=1)
    bo = jnp.zeros((8, H), f32).at[0, :A].set(b_out)
    out = _head(h, seq3, wo, bo)
    return out[:, 0, 0]


# speedup vs baseline: 12.5215x; 1.0099x over previous
import functools

import jax
import jax.numpy as jnp
from jax import lax
from jax.experimental import pallas as pl
from jax.experimental.pallas import tpu as pltpu
from jax.experimental.pallas import tpu_sc as plsc

B, L, K, H, A, NRBF, NL = 4, 512, 48, 128, 21, 16, 3
N = B * L
E_TOT = N * K
NW = 32
ROWS_W = E_TOT // NW
CHUNK = 128
NCH = ROWS_W // CHUNK
NG = 3
NGRP = NCH // NG
NT = 256

_SIGMA = (22.0 - 2.0) / NRBF


def _ln(x):
    mu = x.mean(-1, keepdims=True)
    var = ((x - mu) ** 2).mean(-1, keepdims=True)
    return (x - mu) / jnp.sqrt(var + 1e-5)


RT = 512


def _centers_row():
    return 2.0 + lax.broadcasted_iota(jnp.int32, (1, NRBF), 1).astype(
        jnp.float32) * (20.0 / (NRBF - 1))


def _stage_a_body(ca_s_ref, ca_t_ref, gq_ref, vqc_ref, gidx_ref, dnb_ref,
                  isig_ref):
    b = pl.program_id(0)
    cs = ca_s_ref[0]
    ct = ca_t_ref[0]
    dx = cs[:, 0:1] - ct[0:1, :]
    dy = cs[:, 1:2] - ct[1:2, :]
    dz = cs[:, 2:3] - ct[2:3, :]
    d = jnp.sqrt(dx * dx + dy * dy + dz * dz + 1e-6)
    iot = lax.broadcasted_iota(jnp.int32, (RT, L), 1)
    centers = _centers_row()
    vq_col = vqc_ref[0:1, :].reshape(NRBF, 1)
    cq = vqc_ref[1, 0]
    key = (lax.bitcast_convert_type(d, jnp.int32) & jnp.int32(~511)) | iot
    dcols, icols = [], []
    for _ in range(K):
        kmin = jnp.min(key, axis=1, keepdims=True)
        im = kmin & 511
        dcols.append(lax.bitcast_convert_type(kmin - im, jnp.float32))
        icols.append(im)
        key = jnp.where(iot == im, jnp.int32(0x7FFFFFFF), key)
    gidx_ref[0] = jnp.concatenate(icols, axis=1) + b * L
    dnb_ref[0] = jnp.concatenate(dcols, axis=1)
    dtall = jnp.concatenate(dcols, axis=0)
    r = jnp.exp(-(((dtall - centers) / _SIGMA) ** 2))
    t = jnp.dot(r, gq_ref[...], preferred_element_type=jnp.float32)
    var = jnp.sum(t * r, axis=1, keepdims=True)
    var = var + 2.0 * jnp.dot(r, vq_col,
                              preferred_element_type=jnp.float32) + cq
    istall = 1.0 / jnp.sqrt(var + 1e-5)
    isig_ref[0] = jnp.concatenate(
        [istall[k * RT:(k + 1) * RT] for k in range(K)], axis=1)


def _stage_a(ca_s, ca_t, gq, vqc):
    return pl.pallas_call(
        _stage_a_body,
        grid=(B, L // RT),
        in_specs=[
            pl.BlockSpec((1, RT, 8), lambda b, t: (b, t, 0)),
            pl.BlockSpec((1, 8, L), lambda b, t: (b, 0, 0)),
            pl.BlockSpec((NRBF, NRBF), lambda b, t: (0, 0)),
            pl.BlockSpec((8, NRBF), lambda b, t: (0, 0)),
        ],
        out_specs=[
            pl.BlockSpec((1, RT, K), lambda b, t: (b, t, 0)),
            pl.BlockSpec((1, RT, K), lambda b, t: (b, t, 0)),
            pl.BlockSpec((1, RT, K), lambda b, t: (b, t, 0)),
        ],
        out_shape=[
            jax.ShapeDtypeStruct((B, L, K), jnp.int32),
            jax.ShapeDtypeStruct((B, L, K), jnp.float32),
            jax.ShapeDtypeStruct((B, L, K), jnp.float32),
        ],
    )(ca_s, ca_t, gq, vqc)


def _prep_body(seq_ref, t_ref, out_ref):
    sc = seq_ref[0]
    iot = lax.broadcasted_iota(jnp.int32, (L, 32), 1)
    oh = (sc == iot).astype(jnp.float32)
    out_ref[0, 0] = jnp.dot(oh, t_ref[0],
                            preferred_element_type=jnp.float32)


def _prep(seq3, tpad):
    return pl.pallas_call(
        _prep_body,
        grid=(NL, B),
        in_specs=[
            pl.BlockSpec((1, L, 1), lambda l, b: (b, 0, 0)),
            pl.BlockSpec((1, 32, H), lambda l, b: (l, 0, 0)),
        ],
        out_specs=pl.BlockSpec((1, 1, L, H), lambda l, b: (l, b, 0, 0)),
        out_shape=jax.ShapeDtypeStruct((NL, B, L, H), jnp.float32),
    )(seq3, tpad)


def _layer_body(has_g, has_hs, has_p, *refs):
    i = 0
    if has_g:
        h_ref, g_ref = refs[0], refs[1]
        i = 2
    dnb_ref, isig_ref, m_ref, cb_ref, w1a_ref, w2_ref = refs[i:i + 6]
    i += 6
    if has_p:
        w1bn_ref = refs[i]
        i += 1
    if has_hs:
        hsn_ref = refs[i]
        i += 1
    hn_ref = refs[i]
    if has_p:
        p_ref = refs[i + 1]

    dnb = dnb_ref[...]
    isg = isig_ref[...]
    centers = _centers_row()
    rfeat = jnp.concatenate(
        [jnp.exp(-(((dnb[:, k:k + 1] - centers) / _SIGMA) ** 2))
         for k in range(K)], axis=0)
    stall = jnp.concatenate(
        [isg[:, k:k + 1] for k in range(K)], axis=0)
    rtall = jnp.dot(rfeat, m_ref[...],
                    preferred_element_type=jnp.float32)
    mfull = (rtall + cb_ref[0:1, :]) * stall
    if has_g:
        h = h_ref[...]
        hi = jnp.dot(h, w1a_ref[...],
                     preferred_element_type=jnp.float32) + cb_ref[1:2, :]
        g2 = g_ref[...].reshape(K * NT, H)
        mfull = mfull + g2 + jnp.concatenate([hi] * K, axis=0)
    else:
        mfull = mfull + cb_ref[1:2, :]
    gf = jax.nn.gelu(mfull)
    s = gf[0:NT]
    for k in range(1, K):
        s = s + gf[k * NT:(k + 1) * NT]
    s = s / jnp.float32(K)
    u = jnp.dot(s, w2_ref[...], preferred_element_type=jnp.float32)
    u = u + cb_ref[2:3, :]
    hn = _ln(h + u) if has_g else _ln(u)
    hn_ref[...] = hn
    if has_p:
        p = jnp.dot(hn, w1bn_ref[...], preferred_element_type=jnp.float32)
        if has_hs:
            p = p + hsn_ref[...]
        p_ref[...] = p


TPB = L // NT


def _layer(h, g, dnb, isig, m, cb, w1a, w2, w1bn, hsn):
    has_g = g is not None
    has_p = w1bn is not None
    has_hs = hsn is not None
    steps = N // NT
    node2 = pl.BlockSpec((NT, H), lambda i: (i, 0))
    const = lambda shape: pl.BlockSpec(shape, lambda i: (0, 0))
    in_specs, args = [], []
    if has_g:
        in_specs += [node2,
                     pl.BlockSpec((1, K, NT, H),
                                  lambda i: (i // TPB, 0, i % TPB, 0))]
        args += [h, g]
    in_specs += [pl.BlockSpec((NT, K), lambda i: (i, 0)),
                 pl.BlockSpec((NT, K), lambda i: (i, 0)),
                 const((NRBF, H)), const((8, H)), const((H, H)),
                 const((H, H))]
    args += [dnb, isig, m, cb, w1a, w2]
    if has_p:
        in_specs.append(const((H, H)))
        args.append(w1bn)
    if has_hs:
        in_specs.append(node2)
        args.append(hsn)
    out_specs = [node2]
    out_shape = [jax.ShapeDtypeStruct((N, H), jnp.float32)]
    if has_p:
        out_specs.append(node2)
        out_shape.append(jax.ShapeDtypeStruct((N, H), jnp.float32))
    outs = pl.pallas_call(
        functools.partial(_layer_body, has_g, has_hs, has_p),
        grid=(steps,),
        in_specs=in_specs,
        out_specs=out_specs,
        out_shape=out_shape,
    )(*args)
    return outs if has_p else (outs[0], None)


def _head_body(h_ref, seq_ref, wo_ref, bo_ref, out_ref):
    logits = jnp.dot(h_ref[...], wo_ref[...],
                     preferred_element_type=jnp.float32) + bo_ref[0:1, :]
    lane = lax.broadcasted_iota(jnp.int32, (L, H), 1)
    valid = lane < A
    neg = jnp.float32(-1e30)
    mx = jnp.max(jnp.where(valid, logits, neg), axis=1, keepdims=True)
    ex = jnp.where(valid, jnp.exp(logits - mx), 0.0)
    lse = jnp.log(jnp.sum(ex, axis=1, keepdims=True)) + mx
    sc = seq_ref[0]
    sel = jnp.sum(jnp.where(lane == sc, logits, 0.0), axis=1,
                  keepdims=True)
    val = jnp.sum(sel - lse) / jnp.float32(L)
    out_ref[...] = jnp.full((1, 1, H), val, jnp.float32)


def _head(h, seq3, wo, bo):
    return pl.pallas_call(
        _head_body,
        grid=(B,),
        in_specs=[
            pl.BlockSpec((L, H), lambda b: (b, 0)),
            pl.BlockSpec((1, L, 1), lambda b: (b, 0, 0)),
            pl.BlockSpec((H, H), lambda b: (0, 0)),
            pl.BlockSpec((8, H), lambda b: (0, 0)),
        ],
        out_specs=pl.BlockSpec((1, 1, H), lambda b: (b, 0, 0)),
        out_shape=jax.ShapeDtypeStruct((B, 1, H), jnp.float32),
    )(h, seq3, wo, bo)


def _sc_gather(table, gidx3):
    mesh = plsc.VectorSubcoreMesh(core_axis_name="c", subcore_axis_name="s")

    @functools.partial(
        pl.kernel, mesh=mesh,
        out_type=jax.ShapeDtypeStruct((NW, NCH, CHUNK, H), jnp.float32),
        scratch_types=[
            pltpu.VMEM((NCH, CHUNK), jnp.int32),
            pltpu.VMEM((CHUNK, H), jnp.float32),
            pltpu.VMEM((CHUNK, H), jnp.float32),
            pltpu.SemaphoreType.DMA,
            pltpu.SemaphoreType.DMA,
        ],
    )
    def k(table_hbm, idx_hbm, out_hbm, idx_v, rows0, rows1, sem0, sem1):
        w = lax.axis_index("s") * 2 + lax.axis_index("c")
        pltpu.sync_copy(idx_hbm.at[w], idx_v)
        bufs = (rows0, rows1)
        sems = (sem0, sem1)
        for t in range(2):
            pltpu.make_async_copy(table_hbm.at[idx_v.at[t]], bufs[t],
                                  sems[t]).start()

        def body(i, carry):
            j0 = 2 * i
            for t in range(2):
                j = j0 + t
                pltpu.make_async_copy(table_hbm.at[idx_v.at[j]], bufs[t],
                                      sems[t]).wait()
                pltpu.sync_copy(bufs[t], out_hbm.at[w, j])

                @pl.when(j + 2 < NCH)
                def _():
                    pltpu.make_async_copy(table_hbm.at[idx_v.at[j + 2]],
                                          bufs[t], sems[t]).start()
            return carry

        lax.fori_loop(0, NCH // 2, body, 0)

    return k(table, gidx3)


def _gather_edges(p, gidx3):
    out = _sc_gather(p, gidx3)
    return out.reshape(B, K, L, H)


def kernel(seq_samples, structure, s_emb, W_e, b_e, enc_W1, enc_b1, enc_W2,
           enc_b2, dec_W1, dec_b1, dec_W2, dec_b2, W_out, b_out):
    f32 = jnp.float32
    seq3 = seq_samples.astype(jnp.int32)[..., None]
    ca = structure[:, :, 1, :]
    ca_s = jnp.concatenate([ca, jnp.zeros((B, L, 5), f32)], axis=2)
    ca_t = jnp.swapaxes(ca_s, 1, 2)

    wbar = W_e.mean(axis=1, keepdims=True)
    wc = W_e - wbar
    bc = b_e - b_e.mean()
    gq = jnp.dot(wc, wc.T) / H
    vq = jnp.dot(wc, bc) / H
    vqc = jnp.zeros((8, NRBF), f32).at[0].set(vq).at[1, 0].set(
        jnp.dot(bc, bc) / H)

    gidx, dnb, isig = _stage_a(ca_s, ca_t, gq, vqc)
    gidx3 = gidx.transpose(0, 2, 1).reshape(NW, NCH, CHUNK)
    dnb = dnb.reshape(N, K)
    isig = isig.reshape(N, K)

    eW1a = [enc_W1[l][:H] for l in range(NL)]
    eW1b = [enc_W1[l][H:2 * H] for l in range(NL)]
    eM = [jnp.dot(wc, enc_W1[l][2 * H:]) for l in range(NL)]
    ec = [jnp.dot(bc, enc_W1[l][2 * H:]) for l in range(NL)]
    dW1a = [dec_W1[l][:H] for l in range(NL)]
    dW1b = [dec_W1[l][H:2 * H] for l in range(NL)]
    dT = jnp.stack([jnp.dot(s_emb, dec_W1[l][2 * H:3 * H])
                    for l in range(NL)])
    dM = [jnp.dot(wc, dec_W1[l][3 * H:]) for l in range(NL)]
    dc = [jnp.dot(bc, dec_W1[l][3 * H:]) for l in range(NL)]

    def cbpack(c, b1, b2):
        z = jnp.zeros((8, H), f32)
        return z.at[0].set(c).at[1].set(b1).at[2].set(b2)

    ecb = [cbpack(ec[l], enc_b1[l], enc_b2[l]) for l in range(NL)]
    dcb = [cbpack(dc[l], dec_b1[l], dec_b2[l]) for l in range(NL)]

    tpad = jnp.concatenate([dT, jnp.zeros((NL, 32 - A, H), f32)], axis=1)
    hs3 = _prep(seq3, tpad).reshape(NL, N, H)

    h, p = _layer(None, None, dnb, isig, eM[0], ecb[0], eW1a[0], enc_W2[0],
                  eW1b[1], None)
    g = _gather_edges(p, gidx3)
    h, p = _layer(h, g, dnb, isig, eM[1], ecb[1], eW1a[1], enc_W2[1],
                  eW1b[2], None)
    g = _gather_edges(p, gidx3)
    h, p = _layer(h, g, dnb, isig, eM[2], ecb[2], eW1a[2], enc_W2[2],
                  dW1b[0], hs3[0])
    g = _gather_edges(p, gidx3)
    h, p = _layer(h, g, dnb, isig, dM[0], dcb[0], dW1a[0], dec_W2[0],
                  dW1b[1], hs3[1])
    g = _gather_edges(p, gidx3)
    h, p = _layer(h, g, dnb, isig, dM[1], dcb[1], dW1a[1], dec_W2[1],
                  dW1b[2], hs3[2])
    g = _gather_edges(p, gidx3)
    h, _ = _layer(h, g, dnb, isig, dM[2], dcb[2], dW1a[2], dec_W2[2],
                  None, None)

    wo = jnp.concatenate([W_out, jnp.zeros((H, H - A), f32)], axis=1)
    bo = jnp.zeros((8, H), f32).at[0, :A].set(b_out)
    out = _head(h, seq3, wo, bo)
    return out[:, 0, 0]


# speedup vs baseline: 12.5370x; 1.0012x over previous
import functools

import jax
import jax.numpy as jnp
from jax import lax
from jax.experimental import pallas as pl
from jax.experimental.pallas import tpu as pltpu
from jax.experimental.pallas import tpu_sc as plsc

B, L, K, H, A, NRBF, NL = 4, 512, 48, 128, 21, 16, 3
N = B * L
E_TOT = N * K
NW = 32
ROWS_W = E_TOT // NW
CHUNK = 128
NCH = ROWS_W // CHUNK
NT = 256

_SIGMA = (22.0 - 2.0) / NRBF


def _ln(x):
    mu = x.mean(-1, keepdims=True)
    var = ((x - mu) ** 2).mean(-1, keepdims=True)
    return (x - mu) / jnp.sqrt(var + 1e-5)


RT = 512


def _centers_row():
    return 2.0 + lax.broadcasted_iota(jnp.int32, (1, NRBF), 1).astype(
        jnp.float32) * (20.0 / (NRBF - 1))


def _stage_a_body(ca_s_ref, ca_t_ref, gq_ref, vqc_ref, gidx_ref, dnb_ref,
                  isig_ref):
    b = pl.program_id(0)
    cs = ca_s_ref[0]
    ct = ca_t_ref[0]
    dx = cs[:, 0:1] - ct[0:1, :]
    dy = cs[:, 1:2] - ct[1:2, :]
    dz = cs[:, 2:3] - ct[2:3, :]
    d = jnp.sqrt(dx * dx + dy * dy + dz * dz + 1e-6)
    iot = lax.broadcasted_iota(jnp.int32, (RT, L), 1)
    centers = _centers_row()
    vq_col = vqc_ref[0:1, :].reshape(NRBF, 1)
    cq = vqc_ref[1, 0]
    key = (lax.bitcast_convert_type(d, jnp.int32) & jnp.int32(~511)) | iot
    dcols, icols = [], []
    for _ in range(K):
        kmin = jnp.min(key, axis=1, keepdims=True)
        im = kmin & 511
        dcols.append(lax.bitcast_convert_type(kmin - im, jnp.float32))
        icols.append(im)
        key = jnp.where(iot == im, jnp.int32(0x7FFFFFFF), key)
    gidx_ref[0] = jnp.concatenate(icols, axis=1) + b * L
    dnb_ref[0] = jnp.concatenate(dcols, axis=1)
    dtall = jnp.concatenate(dcols, axis=0)
    r = jnp.exp(-(((dtall - centers) / _SIGMA) ** 2))
    t = jnp.dot(r, gq_ref[...], preferred_element_type=jnp.float32)
    var = jnp.sum(t * r, axis=1, keepdims=True)
    var = var + 2.0 * jnp.dot(r, vq_col,
                              preferred_element_type=jnp.float32) + cq
    istall = 1.0 / jnp.sqrt(var + 1e-5)
    isig_ref[0] = jnp.concatenate(
        [istall[k * RT:(k + 1) * RT] for k in range(K)], axis=1)


def _stage_a(ca_s, ca_t, gq, vqc):
    return pl.pallas_call(
        _stage_a_body,
        grid=(B, L // RT),
        in_specs=[
            pl.BlockSpec((1, RT, 8), lambda b, t: (b, t, 0)),
            pl.BlockSpec((1, 8, L), lambda b, t: (b, 0, 0)),
            pl.BlockSpec((NRBF, NRBF), lambda b, t: (0, 0)),
            pl.BlockSpec((8, NRBF), lambda b, t: (0, 0)),
        ],
        out_specs=[
            pl.BlockSpec((1, RT, K), lambda b, t: (b, t, 0)),
            pl.BlockSpec((1, RT, K), lambda b, t: (b, t, 0)),
            pl.BlockSpec((1, RT, K), lambda b, t: (b, t, 0)),
        ],
        out_shape=[
            jax.ShapeDtypeStruct((B, L, K), jnp.int32),
            jax.ShapeDtypeStruct((B, L, K), jnp.float32),
            jax.ShapeDtypeStruct((B, L, K), jnp.float32),
        ],
    )(ca_s, ca_t, gq, vqc)


def _prep_body(seq_ref, t_ref, out_ref):
    sc = seq_ref[0]
    iot = lax.broadcasted_iota(jnp.int32, (L, 32), 1)
    oh = (sc == iot).astype(jnp.float32)
    out_ref[0, 0] = jnp.dot(oh, t_ref[0],
                            preferred_element_type=jnp.float32)


def _prep(seq3, tpad):
    return pl.pallas_call(
        _prep_body,
        grid=(NL, B),
        in_specs=[
            pl.BlockSpec((1, L, 1), lambda l, b: (b, 0, 0)),
            pl.BlockSpec((1, 32, H), lambda l, b: (l, 0, 0)),
        ],
        out_specs=pl.BlockSpec((1, 1, L, H), lambda l, b: (l, b, 0, 0)),
        out_shape=jax.ShapeDtypeStruct((NL, B, L, H), jnp.float32),
    )(seq3, tpad)


def _layer_body(has_g, has_hs, has_p, *refs):
    i = 0
    if has_g:
        h_ref, g_ref = refs[0], refs[1]
        i = 2
    dnb_ref, isig_ref, m_ref, cb_ref, w1a_ref, w2_ref = refs[i:i + 6]
    i += 6
    if has_p:
        w1bn_ref = refs[i]
        i += 1
    if has_hs:
        hsn_ref = refs[i]
        i += 1
    hn_ref = refs[i]
    if has_p:
        p_ref = refs[i + 1]

    dnb = dnb_ref[...]
    isg = isig_ref[...]
    centers = _centers_row()
    rfeat = jnp.concatenate(
        [jnp.exp(-(((dnb[:, k:k + 1] - centers) / _SIGMA) ** 2))
         for k in range(K)], axis=0)
    stall = jnp.concatenate(
        [isg[:, k:k + 1] for k in range(K)], axis=0)
    rtall = jnp.dot(rfeat, m_ref[...],
                    preferred_element_type=jnp.float32)
    mfull = (rtall + cb_ref[0:1, :]) * stall
    if has_g:
        h = h_ref[...]
        hi = jnp.dot(h, w1a_ref[...],
                     preferred_element_type=jnp.float32) + cb_ref[1:2, :]
        g2 = g_ref[...].reshape(K * NT, H)
        mfull = mfull + g2 + jnp.concatenate([hi] * K, axis=0)
    else:
        mfull = mfull + cb_ref[1:2, :]
    gf = jax.nn.gelu(mfull)
    s = gf[0:NT]
    for k in range(1, K):
        s = s + gf[k * NT:(k + 1) * NT]
    s = s / jnp.float32(K)
    u = jnp.dot(s, w2_ref[...], preferred_element_type=jnp.float32)
    u = u + cb_ref[2:3, :]
    hn = _ln(h + u) if has_g else _ln(u)
    hn_ref[...] = hn
    if has_p:
        p = jnp.dot(hn, w1bn_ref[...], preferred_element_type=jnp.float32)
        if has_hs:
            p = p + hsn_ref[...]
        p_ref[...] = p


TPB = L // NT


def _layer(h, g, dnb, isig, m, cb, w1a, w2, w1bn, hsn):
    has_g = g is not None
    has_p = w1bn is not None
    has_hs = hsn is not None
    steps = N // NT
    node2 = pl.BlockSpec((NT, H), lambda i: (i, 0))
    const = lambda shape: pl.BlockSpec(shape, lambda i: (0, 0))
    in_specs, args = [], []
    if has_g:
        in_specs += [node2,
                     pl.BlockSpec((1, K, NT, H),
                                  lambda i: (i // TPB, 0, i % TPB, 0))]
        args += [h, g]
    in_specs += [pl.BlockSpec((NT, K), lambda i: (i, 0)),
                 pl.BlockSpec((NT, K), lambda i: (i, 0)),
                 const((NRBF, H)), const((8, H)), const((H, H)),
                 const((H, H))]
    args += [dnb, isig, m, cb, w1a, w2]
    if has_p:
        in_specs.append(const((H, H)))
        args.append(w1bn)
    if has_hs:
        in_specs.append(node2)
        args.append(hsn)
    out_specs = [node2]
    out_shape = [jax.ShapeDtypeStruct((N, H), jnp.float32)]
    if has_p:
        out_specs.append(node2)
        out_shape.append(jax.ShapeDtypeStruct((N, H), jnp.float32))
    outs = pl.pallas_call(
        functools.partial(_layer_body, has_g, has_hs, has_p),
        grid=(steps,),
        in_specs=in_specs,
        out_specs=out_specs,
        out_shape=out_shape,
    )(*args)
    return outs if has_p else (outs[0], None)


def _head_body(h_ref, seq_ref, wo_ref, bo_ref, out_ref):
    logits = jnp.dot(h_ref[...], wo_ref[...],
                     preferred_element_type=jnp.float32) + bo_ref[0:1, :]
    lane = lax.broadcasted_iota(jnp.int32, (L, H), 1)
    valid = lane < A
    neg = jnp.float32(-1e30)
    mx = jnp.max(jnp.where(valid, logits, neg), axis=1, keepdims=True)
    ex = jnp.where(valid, jnp.exp(logits - mx), 0.0)
    lse = jnp.log(jnp.sum(ex, axis=1, keepdims=True)) + mx
    sc = seq_ref[0]
    sel = jnp.sum(jnp.where(lane == sc, logits, 0.0), axis=1,
                  keepdims=True)
    val = jnp.sum(sel - lse) / jnp.float32(L)
    out_ref[...] = jnp.full((1, 1, H), val, jnp.float32)


def _head(h, seq3, wo, bo):
    return pl.pallas_call(
        _head_body,
        grid=(B,),
        in_specs=[
            pl.BlockSpec((L, H), lambda b: (b, 0)),
            pl.BlockSpec((1, L, 1), lambda b: (b, 0, 0)),
            pl.BlockSpec((H, H), lambda b: (0, 0)),
            pl.BlockSpec((8, H), lambda b: (0, 0)),
        ],
        out_specs=pl.BlockSpec((1, 1, H), lambda b: (b, 0, 0)),
        out_shape=jax.ShapeDtypeStruct((B, 1, H), jnp.float32),
    )(h, seq3, wo, bo)


def _sc_gather(table, gidx3):
    mesh = plsc.VectorSubcoreMesh(core_axis_name="c", subcore_axis_name="s")

    @functools.partial(
        pl.kernel, mesh=mesh,
        out_type=jax.ShapeDtypeStruct((NW, NCH, CHUNK, H), jnp.float32),
        scratch_types=[
            pltpu.VMEM((NCH, CHUNK), jnp.int32),
            pltpu.VMEM((CHUNK, H), jnp.float32),
            pltpu.VMEM((CHUNK, H), jnp.float32),
            pltpu.SemaphoreType.DMA,
            pltpu.SemaphoreType.DMA,
        ],
    )
    def k(table_hbm, idx_hbm, out_hbm, idx_v, rows0, rows1, sem0, sem1):
        w = lax.axis_index("s") * 2 + lax.axis_index("c")
        pltpu.sync_copy(idx_hbm.at[w], idx_v)
        bufs = (rows0, rows1)
        sems = (sem0, sem1)
        for t in range(2):
            pltpu.make_async_copy(table_hbm.at[idx_v.at[t]], bufs[t],
                                  sems[t]).start()

        def body(i, carry):
            j0 = 2 * i
            for t in range(2):
                j = j0 + t
                pltpu.make_async_copy(table_hbm.at[idx_v.at[j]], bufs[t],
                                      sems[t]).wait()
                pltpu.sync_copy(bufs[t], out_hbm.at[w, j])

                @pl.when(j + 2 < NCH)
                def _():
                    pltpu.make_async_copy(table_hbm.at[idx_v.at[j + 2]],
                                          bufs[t], sems[t]).start()
            return carry

        lax.fori_loop(0, NCH // 2, body, 0)

    return k(table, gidx3)


def _gather_edges(p, gidx3):
    out = _sc_gather(p, gidx3)
    return out.reshape(B, K, L, H)


def kernel(seq_samples, structure, s_emb, W_e, b_e, enc_W1, enc_b1, enc_W2,
           enc_b2, dec_W1, dec_b1, dec_W2, dec_b2, W_out, b_out):
    f32 = jnp.float32
    seq3 = seq_samples.astype(jnp.int32)[..., None]
    ca = structure[:, :, 1, :]
    ca_s = jnp.concatenate([ca, jnp.zeros((B, L, 5), f32)], axis=2)
    ca_t = jnp.swapaxes(ca_s, 1, 2)

    wbar = W_e.mean(axis=1, keepdims=True)
    wc = W_e - wbar
    bc = b_e - b_e.mean()
    gq = jnp.dot(wc, wc.T) / H
    vq = jnp.dot(wc, bc) / H
    vqc = jnp.zeros((8, NRBF), f32).at[0].set(vq).at[1, 0].set(
        jnp.dot(bc, bc) / H)

    gidx, dnb, isig = _stage_a(ca_s, ca_t, gq, vqc)
    gidx3 = gidx.transpose(0, 2, 1).reshape(NW, NCH, CHUNK)
    dnb = dnb.reshape(N, K)
    isig = isig.reshape(N, K)

    eW1a = [enc_W1[l][:H] for l in range(NL)]
    eW1b = [enc_W1[l][H:2 * H] for l in range(NL)]
    eM = [jnp.dot(wc, enc_W1[l][2 * H:]) for l in range(NL)]
    ec = [jnp.dot(bc, enc_W1[l][2 * H:]) for l in range(NL)]
    dW1a = [dec_W1[l][:H] for l in range(NL)]
    dW1b = [dec_W1[l][H:2 * H] for l in range(NL)]
    dT = jnp.stack([jnp.dot(s_emb, dec_W1[l][2 * H:3 * H])
                    for l in range(NL)])
    dM = [jnp.dot(wc, dec_W1[l][3 * H:]) for l in range(NL)]
    dc = [jnp.dot(bc, dec_W1[l][3 * H:]) for l in range(NL)]

    def cbpack(c, b1, b2):
        z = jnp.zeros((8, H), f32)
        return z.at[0].set(c).at[1].set(b1).at[2].set(b2)

    ecb = [cbpack(ec[l], enc_b1[l], enc_b2[l]) for l in range(NL)]
    dcb = [cbpack(dc[l], dec_b1[l], dec_b2[l]) for l in range(NL)]

    tpad = jnp.concatenate([dT, jnp.zeros((NL, 32 - A, H), f32)], axis=1)
    hs3 = _prep(seq3, tpad).reshape(NL, N, H)

    h, p = _layer(None, None, dnb, isig, eM[0], ecb[0], eW1a[0], enc_W2[0],
                  eW1b[1], None)
    g = _gather_edges(p, gidx3)
    h, p = _layer(h, g, dnb, isig, eM[1], ecb[1], eW1a[1], enc_W2[1],
                  eW1b[2], None)
    g = _gather_edges(p, gidx3)
    h, p = _layer(h, g, dnb, isig, eM[2], ecb[2], eW1a[2], enc_W2[2],
                  dW1b[0], hs3[0])
    g = _gather_edges(p, gidx3)
    h, p = _layer(h, g, dnb, isig, dM[0], dcb[0], dW1a[0], dec_W2[0],
                  dW1b[1], hs3[1])
    g = _gather_edges(p, gidx3)
    h, p = _layer(h, g, dnb, isig, dM[1], dcb[1], dW1a[1], dec_W2[1],
                  dW1b[2], hs3[2])
    g = _gather_edges(p, gidx3)
    h, _ = _layer(h, g, dnb, isig, dM[2], dcb[2], dW1a[2], dec_W2[2],
                  None, None)

    wo = jnp.concatenate([W_out, jnp.zeros((H, H - A), f32)], axis=1)
    bo = jnp.zeros((8, H), f32).at[0, :A].set(b_out)
    out = _head(h, seq3, wo, bo)
    return out[:, 0, 0]
